# baseline (speedup 1.0000x reference)
import functools
import numpy as np
import jax
import jax.numpy as jnp
from jax import lax
from jax.experimental import pallas as pl
from jax.experimental.pallas import tpu as pltpu
from jax.experimental.pallas import tpu_sc as plsc

N_ATOMS = 10000
N_CG = 1000
N_EDGES = 320000
N_CG_EDGES = 32000
N_ATOM_BASIS = 128
N_FILTERS = 128
N_GAUSSIANS = 50
N_CONV = 3
CUTOFF = 5.0

_OFFSETS = np.linspace(0.0, CUTOFF, N_GAUSSIANS).astype(np.float32)
_WIDTH = float(_OFFSETS[1] - _OFFSETS[0])
_COEFF = -0.5 / _WIDTH**2
_LOG2 = float(np.log(2.0))


def _ssp(x):
    return jnp.logaddexp(x, 0.0) - _LOG2


def _embed_body(z_ref, emb_ref, out_ref):
    z = z_ref[...]
    oh = (z == jax.lax.broadcasted_iota(jnp.int32, (1, 100), 1)).astype(jnp.float32)
    out_ref[...] = jnp.dot(oh, emb_ref[...], preferred_element_type=jnp.float32)


def _embed(z2d, embed):
    n = z2d.shape[0]
    blk = 1000
    return pl.pallas_call(
        _embed_body,
        grid=(n // blk,),
        in_specs=[
            pl.BlockSpec((blk, 1), lambda i: (i, 0)),
            pl.BlockSpec((100, N_ATOM_BASIS), lambda i: (0, 0)),
        ],
        out_specs=pl.BlockSpec((blk, N_ATOM_BASIS), lambda i: (i, 0)),
        out_shape=jax.ShapeDtypeStruct((n, N_ATOM_BASIS), jnp.float32),
    )(z2d, embed)


def _counts_body(m_ref, out_ref):
    i = pl.program_id(0)
    m = m_ref[...]
    oh = (m == jax.lax.broadcasted_iota(jnp.int32, (1, N_CG), 1)).astype(jnp.float32)
    c = jnp.sum(oh, axis=0, keepdims=True)

    @pl.when(i == 0)
    def _():
        out_ref[...] = jnp.zeros_like(out_ref)

    out_ref[...] += c


def _counts(m2d):
    n = m2d.shape[0]
    blk = 1000
    return pl.pallas_call(
        _counts_body,
        grid=(n // blk,),
        in_specs=[pl.BlockSpec((blk, 1), lambda i: (i, 0))],
        out_specs=pl.BlockSpec((1, N_CG), lambda i: (0, 0)),
        out_shape=jax.ShapeDtypeStruct((1, N_CG), jnp.float32),
    )(m2d)


def _edge_filter_body(g_ref, W1_ref, b1_ref, W2_ref, b2_ref, out_ref):
    diff = g_ref[0] - g_ref[1]
    d = jnp.sqrt(jnp.sum(diff * diff, axis=1, keepdims=True))
    offs = jax.lax.broadcasted_iota(jnp.int32, (1, N_GAUSSIANS), 1).astype(jnp.float32) * (CUTOFF / (N_GAUSSIANS - 1))
    g = jnp.exp(_COEFF * (d - offs) ** 2)
    h = _ssp(jnp.dot(g, W1_ref[...], preferred_element_type=jnp.float32) + b1_ref[...])
    out_ref[...] = jnp.dot(h, W2_ref[...], preferred_element_type=jnp.float32) + b2_ref[...]


def _edge_filter(gpair, W1, b1, W2, b2):
    e = gpair.shape[1]
    blk = 2000
    return pl.pallas_call(
        _edge_filter_body,
        grid=(e // blk,),
        in_specs=[
            pl.BlockSpec((2, blk, 8), lambda i: (0, i, 0)),
            pl.BlockSpec((N_GAUSSIANS, N_GAUSSIANS), lambda i: (0, 0)),
            pl.BlockSpec((1, N_GAUSSIANS), lambda i: (0, 0)),
            pl.BlockSpec((N_GAUSSIANS, N_FILTERS), lambda i: (0, 0)),
            pl.BlockSpec((1, N_FILTERS), lambda i: (0, 0)),
        ],
        out_specs=pl.BlockSpec((blk, N_FILTERS), lambda i: (i, 0)),
        out_shape=jax.ShapeDtypeStruct((e, N_FILTERS), jnp.float32),
    )(gpair, W1, b1, W2, b2)


def _rn_body(s_ref, W_ref, b_ref, out_ref):
    out_ref[...] = jnp.dot(s_ref[...], W_ref[...], preferred_element_type=jnp.float32) + b_ref[...]


def _rn(s, W, b):
    n = s.shape[0]
    blk = min(n, 2000)
    return pl.pallas_call(
        _rn_body,
        grid=(n // blk,),
        in_specs=[
            pl.BlockSpec((blk, N_ATOM_BASIS), lambda i: (i, 0)),
            pl.BlockSpec((N_ATOM_BASIS, N_FILTERS), lambda i: (0, 0)),
            pl.BlockSpec((1, N_FILTERS), lambda i: (0, 0)),
        ],
        out_specs=pl.BlockSpec((blk, N_FILTERS), lambda i: (i, 0)),
        out_shape=jax.ShapeDtypeStruct((n, N_FILTERS), jnp.float32),
    )(s, W, b)


def _update_body(s_ref, a0_ref, a1_ref, W1_ref, b1_ref, W2_ref, b2_ref, out_ref):
    agg = a0_ref[...] + a1_ref[...]
    h = _ssp(jnp.dot(agg, W1_ref[...], preferred_element_type=jnp.float32) + b1_ref[...])
    out_ref[...] = s_ref[...] + jnp.dot(h, W2_ref[...], preferred_element_type=jnp.float32) + b2_ref[...]


def _update(s, a0, a1, Wu1, bu1, Wu2, bu2):
    n = s.shape[0]
    blk = min(n, 2000)
    return pl.pallas_call(
        _update_body,
        grid=(n // blk,),
        in_specs=[
            pl.BlockSpec((blk, N_ATOM_BASIS), lambda i: (i, 0)),
            pl.BlockSpec((blk, N_FILTERS), lambda i: (i, 0)),
            pl.BlockSpec((blk, N_FILTERS), lambda i: (i, 0)),
            pl.BlockSpec((N_FILTERS, N_ATOM_BASIS), lambda i: (0, 0)),
            pl.BlockSpec((1, N_ATOM_BASIS), lambda i: (0, 0)),
            pl.BlockSpec((N_ATOM_BASIS, N_ATOM_BASIS), lambda i: (0, 0)),
            pl.BlockSpec((1, N_ATOM_BASIS), lambda i: (0, 0)),
        ],
        out_specs=pl.BlockSpec((blk, N_ATOM_BASIS), lambda i: (i, 0)),
        out_shape=jax.ShapeDtypeStruct((n, N_ATOM_BASIS), jnp.float32),
    )(s, a0, a1, Wu1, bu1, Wu2, bu2)


def _div_body(S0_ref, S1_ref, c_ref, out_ref):
    c = jnp.maximum(c_ref[...], 1.0)
    out_ref[...] = (S0_ref[...] + S1_ref[...]) / c


def _div(S0, S1, c_col):
    return pl.pallas_call(
        _div_body,
        grid=(1,),
        in_specs=[
            pl.BlockSpec((N_CG, N_ATOM_BASIS), lambda i: (0, 0)),
            pl.BlockSpec((N_CG, N_ATOM_BASIS), lambda i: (0, 0)),
            pl.BlockSpec((N_CG, 1), lambda i: (0, 0)),
        ],
        out_specs=pl.BlockSpec((N_CG, N_ATOM_BASIS), lambda i: (0, 0)),
        out_shape=jax.ShapeDtypeStruct((N_CG, N_ATOM_BASIS), jnp.float32),
    )(S0, S1, c_col)


_NC, _NS = 2, 16
_NW = _NC * _NS


def _sc_mesh():
    return plsc.VectorSubcoreMesh(
        core_axis_name="c", subcore_axis_name="s", num_cores=_NC, num_subcores=_NS)


def _msg_scatter_sc(rn, f, src_ch, dst_ch, zeros, n_nodes, n_chunks):
    N = n_nodes
    BR = 40
    nbl = N // BR
    kz = -(-nbl // _NS)
    kmax = -(-n_chunks // _NW)

    def body(rn_hbm, f_hbm, src_hbm, dst_hbm, z_hbm, out_hbm,
             agg_sh, idx_s, idx_d, f_v, rs_v, rd_v, sem):
        c = lax.axis_index("c")
        s = lax.axis_index("s")
        w = s * _NC + c

        def zero_blk(k2, carry):
            bid = k2 * _NS + s

            @pl.when(bid < nbl)
            def _():
                off = pl.multiple_of(bid * BR, BR)
                pltpu.sync_copy(z_hbm, agg_sh.at[pl.ds(off, BR)])

            return carry

        lax.fori_loop(0, kz, zero_blk, 0)
        plsc.subcore_barrier()

        def chunk(k, carry):
            cid = k * _NW + w

            @pl.when(cid < n_chunks)
            def _():
                pltpu.sync_copy(src_hbm.at[cid], idx_s)
                pltpu.sync_copy(dst_hbm.at[cid], idx_d)
                foff = pl.multiple_of(cid * 128, 128)
                pltpu.sync_copy(f_hbm.at[pl.ds(foff, 128)], f_v)
                d1 = pltpu.async_copy(rn_hbm.at[idx_s], rs_v, sem)
                d2 = pltpu.async_copy(rn_hbm.at[idx_d], rd_v, sem)
                d1.wait()
                d2.wait()

                def row(r, carry2):
                    for j in range(8):
                        sl = pl.ds(j * 16, 16)
                        fv = f_v[r, sl]
                        rs_v[r, sl] = rs_v[r, sl] * fv
                        rd_v[r, sl] = rd_v[r, sl] * fv
                    return carry2

                lax.fori_loop(0, 128, row, 0)
                pltpu.sync_copy(rs_v, agg_sh.at[idx_d], add=True)
                pltpu.sync_copy(rd_v, agg_sh.at[idx_s], add=True)

            return carry

        lax.fori_loop(0, kmax, chunk, 0)
        plsc.subcore_barrier()

        def read_blk(k2, carry):
            bid = k2 * _NS + s

            @pl.when(bid < nbl)
            def _():
                off = pl.multiple_of(bid * BR, BR)
                pltpu.sync_copy(agg_sh.at[pl.ds(off, BR)],
                                out_hbm.at[c, pl.ds(off, BR)])

            return carry

        lax.fori_loop(0, kz, read_blk, 0)

    return pl.kernel(
        body,
        out_type=jax.ShapeDtypeStruct((_NC, N, N_FILTERS), jnp.float32),
        mesh=_sc_mesh(),
        scratch_types=[
            pltpu.VMEM_SHARED((N, N_FILTERS), jnp.float32),
            pltpu.VMEM((128,), jnp.int32),
            pltpu.VMEM((128,), jnp.int32),
            pltpu.VMEM((128, N_FILTERS), jnp.float32),
            pltpu.VMEM((128, N_FILTERS), jnp.float32),
            pltpu.VMEM((128, N_FILTERS), jnp.float32),
            pltpu.SemaphoreType.DMA,
        ],
    )(rn, f, src_ch, dst_ch, zeros)


def _seg_sum_sc(x, map_ch, zeros, n_rows, n_out):
    CH = 80
    n_chunks = n_rows // CH
    kmax = -(-n_chunks // _NW)
    BR = 40
    nbl = n_out // BR
    kz = -(-nbl // _NS)

    def body(x_hbm, map_hbm, z_hbm, out_hbm, acc_sh, idx_v, x_v, sem):
        c = lax.axis_index("c")
        s = lax.axis_index("s")
        w = s * _NC + c

        def zero_blk(k2, carry):
            bid = k2 * _NS + s

            @pl.when(bid < nbl)
            def _():
                off = pl.multiple_of(bid * BR, BR)
                pltpu.sync_copy(z_hbm, acc_sh.at[pl.ds(off, BR)])

            return carry

        lax.fori_loop(0, kz, zero_blk, 0)
        plsc.subcore_barrier()

        def chunk(k, carry):
            cid = k * _NW + w

            @pl.when(cid < n_chunks)
            def _():
                pltpu.sync_copy(map_hbm.at[cid], idx_v)
                roff = pl.multiple_of(cid * CH, CH)
                pltpu.sync_copy(x_hbm.at[pl.ds(roff, CH)], x_v)
                pltpu.sync_copy(x_v, acc_sh.at[idx_v], add=True)

            return carry

        lax.fori_loop(0, kmax, chunk, 0)
        plsc.subcore_barrier()

        def read_blk(k2, carry):
            bid = k2 * _NS + s

            @pl.when(bid < nbl)
            def _():
                off = pl.multiple_of(bid * BR, BR)
                pltpu.sync_copy(acc_sh.at[pl.ds(off, BR)],
                                out_hbm.at[c, pl.ds(off, BR)])

            return carry

        lax.fori_loop(0, kz, read_blk, 0)

    return pl.kernel(
        body,
        out_type=jax.ShapeDtypeStruct((_NC, n_out, N_FILTERS), jnp.float32),
        mesh=_sc_mesh(),
        scratch_types=[
            pltpu.VMEM_SHARED((n_out, N_FILTERS), jnp.float32),
            pltpu.VMEM((CH,), jnp.int32),
            pltpu.VMEM((CH, N_FILTERS), jnp.float32),
            pltpu.SemaphoreType.DMA,
        ],
    )(x, map_ch, zeros)


def _gather_add_sc(x, table, map_ch, n_rows):
    CH = 80
    n_chunks = n_rows // CH
    kmax = -(-n_chunks // _NW)

    def body(x_hbm, tab_hbm, map_hbm, out_hbm, idx_v, x_v, g_v, sem):
        c = lax.axis_index("c")
        s = lax.axis_index("s")
        w = s * _NC + c

        def chunk(k, carry):
            cid = k * _NW + w

            @pl.when(cid < n_chunks)
            def _():
                pltpu.sync_copy(map_hbm.at[cid], idx_v)
                roff = pl.multiple_of(cid * CH, CH)
                d1 = pltpu.async_copy(tab_hbm.at[idx_v], g_v, sem)
                pltpu.sync_copy(x_hbm.at[pl.ds(roff, CH)], x_v)
                d1.wait()

                def row(r, carry2):
                    for j in range(8):
                        sl = pl.ds(j * 16, 16)
                        x_v[r, sl] = x_v[r, sl] + g_v[r, sl]
                    return carry2

                lax.fori_loop(0, CH, row, 0)
                pltpu.sync_copy(x_v, out_hbm.at[pl.ds(roff, CH)])

            return carry

        lax.fori_loop(0, kmax, chunk, 0)

    return pl.kernel(
        body,
        out_type=jax.ShapeDtypeStruct((n_rows, N_FILTERS), jnp.float32),
        mesh=_sc_mesh(),
        scratch_types=[
            pltpu.VMEM((CH,), jnp.int32),
            pltpu.VMEM((CH, N_FILTERS), jnp.float32),
            pltpu.VMEM((CH, N_FILTERS), jnp.float32),
            pltpu.SemaphoreType.DMA,
        ],
    )(x, table, map_ch)


def kernel(z, xyz, cg_xyz, mapping, nbr_list, CG_nbr_list, embed,
           W_ef1, b_ef1, W_ef2, b_ef2, W_nf, b_nf, W_u1, b_u1, W_u2, b_u2):
    z2d = z.astype(jnp.int32)[:, None]
    m2d = mapping.astype(jnp.int32)[:, None]

    src_af = nbr_list[:, 0].astype(jnp.int32)
    dst_af = nbr_list[:, 1].astype(jnp.int32)
    src_cf = CG_nbr_list[:, 0].astype(jnp.int32)
    dst_cf = CG_nbr_list[:, 1].astype(jnp.int32)
    xyz8 = jnp.zeros((N_ATOMS, 8), jnp.float32).at[:, :3].set(xyz.astype(jnp.float32))
    cg8 = jnp.zeros((N_CG, 8), jnp.float32).at[:, :3].set(cg_xyz.astype(jnp.float32))
    gpair_a = jnp.stack([xyz8[src_af], xyz8[dst_af]])
    gpair_c = jnp.stack([cg8[src_cf], cg8[dst_cf]])

    src_a = src_af.reshape(N_EDGES // 128, 128)
    dst_a = dst_af.reshape(N_EDGES // 128, 128)
    src_c = src_cf.reshape(N_CG_EDGES // 128, 128)
    dst_c = dst_cf.reshape(N_CG_EDGES // 128, 128)
    map_ch = mapping.astype(jnp.int32).reshape(N_ATOMS // 80, 80)
    zeros_br = jnp.zeros((40, N_FILTERS), jnp.float32)

    s_i = _embed(z2d, embed)
    counts = _counts(m2d)
    c_col = counts.T

    S_I = None
    for i in range(N_CONV):
        f = _edge_filter(gpair_a, W_ef1[i], b_ef1[i][None, :], W_ef2[i], b_ef2[i][None, :])
        rn = _rn(s_i, W_nf[i], b_nf[i][None, :])
        agg2 = _msg_scatter_sc(rn, f, src_a, dst_a, zeros_br, N_ATOMS, N_EDGES // 128)
        s_i = _update(s_i, agg2[0], agg2[1], W_u1[i], b_u1[i][None, :], W_u2[i], b_u2[i][None, :])

        S_parts = _seg_sum_sc(s_i, map_ch, zeros_br, N_ATOMS, N_CG)
        S_input = _div(S_parts[0], S_parts[1], c_col)
        if i == 0:
            S_I = S_input

        j = N_CONV + i
        fc = _edge_filter(gpair_c, W_ef1[j], b_ef1[j][None, :], W_ef2[j], b_ef2[j][None, :])
        Rn = _rn(S_input, W_nf[j], b_nf[j][None, :])
        Agg2 = _msg_scatter_sc(Rn, fc, src_c, dst_c, zeros_br, N_CG, N_CG_EDGES // 128)
        S_I = _update(S_I, Agg2[0], Agg2[1], W_u1[j], b_u1[j][None, :], W_u2[j], b_u2[j][None, :])

        if i < N_CONV - 1:
            s_i = _gather_add_sc(s_i, S_I, map_ch, N_ATOMS)

    return S_I


# speedup vs baseline: 1.1975x; 1.1975x over previous
import functools
import numpy as np
import jax
import jax.numpy as jnp
from jax import lax
from jax.experimental import pallas as pl
from jax.experimental.pallas import tpu as pltpu
from jax.experimental.pallas import tpu_sc as plsc

N_ATOMS = 10000
N_CG = 1000
N_EDGES = 320000
N_CG_EDGES = 32000
N_ATOM_BASIS = 128
N_FILTERS = 128
N_GAUSSIANS = 50
N_CONV = 3
CUTOFF = 5.0

_OFFSETS = np.linspace(0.0, CUTOFF, N_GAUSSIANS).astype(np.float32)
_WIDTH = float(_OFFSETS[1] - _OFFSETS[0])
_COEFF = -0.5 / _WIDTH**2
_LOG2 = float(np.log(2.0))


def _ssp(x):
    return jnp.logaddexp(x, 0.0) - _LOG2


def _embed_body(z_ref, emb_ref, out_ref):
    z = z_ref[...]
    oh = (z == jax.lax.broadcasted_iota(jnp.int32, (1, 100), 1)).astype(jnp.float32)
    out_ref[...] = jnp.dot(oh, emb_ref[...], preferred_element_type=jnp.float32)


def _embed(z2d, embed):
    n = z2d.shape[0]
    blk = 1000
    return pl.pallas_call(
        _embed_body,
        grid=(n // blk,),
        in_specs=[
            pl.BlockSpec((blk, 1), lambda i: (i, 0)),
            pl.BlockSpec((100, N_ATOM_BASIS), lambda i: (0, 0)),
        ],
        out_specs=pl.BlockSpec((blk, N_ATOM_BASIS), lambda i: (i, 0)),
        out_shape=jax.ShapeDtypeStruct((n, N_ATOM_BASIS), jnp.float32),
    )(z2d, embed)


def _counts_body(m_ref, out_ref):
    i = pl.program_id(0)
    m = m_ref[...]
    oh = (m == jax.lax.broadcasted_iota(jnp.int32, (1, N_CG), 1)).astype(jnp.float32)
    c = jnp.sum(oh, axis=0, keepdims=True)

    @pl.when(i == 0)
    def _():
        out_ref[...] = jnp.zeros_like(out_ref)

    out_ref[...] += c


def _counts(m2d):
    n = m2d.shape[0]
    blk = 1000
    return pl.pallas_call(
        _counts_body,
        grid=(n // blk,),
        in_specs=[pl.BlockSpec((blk, 1), lambda i: (i, 0))],
        out_specs=pl.BlockSpec((1, N_CG), lambda i: (0, 0)),
        out_shape=jax.ShapeDtypeStruct((1, N_CG), jnp.float32),
    )(m2d)


def _edge_filter_body(g_ref, W1_ref, b1_ref, W2_ref, b2_ref, out_ref):
    diff = g_ref[0] - g_ref[1]
    d = jnp.sqrt(jnp.sum(diff * diff, axis=1, keepdims=True))
    offs = jax.lax.broadcasted_iota(jnp.int32, (1, N_GAUSSIANS), 1).astype(jnp.float32) * (CUTOFF / (N_GAUSSIANS - 1))
    g = jnp.exp(_COEFF * (d - offs) ** 2)
    h = _ssp(jnp.dot(g, W1_ref[...], preferred_element_type=jnp.float32) + b1_ref[...])
    out_ref[...] = jnp.dot(h, W2_ref[...], preferred_element_type=jnp.float32) + b2_ref[...]


def _edge_filter(gpair, W1, b1, W2, b2):
    e = gpair.shape[1]
    blk = 2000
    return pl.pallas_call(
        _edge_filter_body,
        grid=(e // blk,),
        in_specs=[
            pl.BlockSpec((2, blk, 8), lambda i: (0, i, 0)),
            pl.BlockSpec((N_GAUSSIANS, N_GAUSSIANS), lambda i: (0, 0)),
            pl.BlockSpec((1, N_GAUSSIANS), lambda i: (0, 0)),
            pl.BlockSpec((N_GAUSSIANS, N_FILTERS), lambda i: (0, 0)),
            pl.BlockSpec((1, N_FILTERS), lambda i: (0, 0)),
        ],
        out_specs=pl.BlockSpec((blk, N_FILTERS), lambda i: (i, 0)),
        out_shape=jax.ShapeDtypeStruct((e, N_FILTERS), jnp.float32),
    )(gpair, W1, b1, W2, b2)


def _rn_body(s_ref, W_ref, b_ref, out_ref):
    out_ref[...] = jnp.dot(s_ref[...], W_ref[...], preferred_element_type=jnp.float32) + b_ref[...]


def _rn(s, W, b):
    n = s.shape[0]
    blk = min(n, 2000)
    return pl.pallas_call(
        _rn_body,
        grid=(n // blk,),
        in_specs=[
            pl.BlockSpec((blk, N_ATOM_BASIS), lambda i: (i, 0)),
            pl.BlockSpec((N_ATOM_BASIS, N_FILTERS), lambda i: (0, 0)),
            pl.BlockSpec((1, N_FILTERS), lambda i: (0, 0)),
        ],
        out_specs=pl.BlockSpec((blk, N_FILTERS), lambda i: (i, 0)),
        out_shape=jax.ShapeDtypeStruct((n, N_FILTERS), jnp.float32),
    )(s, W, b)


def _update_body(s_ref, a0_ref, a1_ref, W1_ref, b1_ref, W2_ref, b2_ref, out_ref):
    agg = a0_ref[...] + a1_ref[...]
    h = _ssp(jnp.dot(agg, W1_ref[...], preferred_element_type=jnp.float32) + b1_ref[...])
    out_ref[...] = s_ref[...] + jnp.dot(h, W2_ref[...], preferred_element_type=jnp.float32) + b2_ref[...]


def _update(s, a0, a1, Wu1, bu1, Wu2, bu2):
    n = s.shape[0]
    blk = min(n, 2000)
    return pl.pallas_call(
        _update_body,
        grid=(n // blk,),
        in_specs=[
            pl.BlockSpec((blk, N_ATOM_BASIS), lambda i: (i, 0)),
            pl.BlockSpec((blk, N_FILTERS), lambda i: (i, 0)),
            pl.BlockSpec((blk, N_FILTERS), lambda i: (i, 0)),
            pl.BlockSpec((N_FILTERS, N_ATOM_BASIS), lambda i: (0, 0)),
            pl.BlockSpec((1, N_ATOM_BASIS), lambda i: (0, 0)),
            pl.BlockSpec((N_ATOM_BASIS, N_ATOM_BASIS), lambda i: (0, 0)),
            pl.BlockSpec((1, N_ATOM_BASIS), lambda i: (0, 0)),
        ],
        out_specs=pl.BlockSpec((blk, N_ATOM_BASIS), lambda i: (i, 0)),
        out_shape=jax.ShapeDtypeStruct((n, N_ATOM_BASIS), jnp.float32),
    )(s, a0, a1, Wu1, bu1, Wu2, bu2)


def _div_body(S0_ref, S1_ref, c_ref, out_ref):
    c = jnp.maximum(c_ref[...], 1.0)
    out_ref[...] = (S0_ref[...] + S1_ref[...]) / c


def _div(S0, S1, c_col):
    return pl.pallas_call(
        _div_body,
        grid=(1,),
        in_specs=[
            pl.BlockSpec((N_CG, N_ATOM_BASIS), lambda i: (0, 0)),
            pl.BlockSpec((N_CG, N_ATOM_BASIS), lambda i: (0, 0)),
            pl.BlockSpec((N_CG, 1), lambda i: (0, 0)),
        ],
        out_specs=pl.BlockSpec((N_CG, N_ATOM_BASIS), lambda i: (0, 0)),
        out_shape=jax.ShapeDtypeStruct((N_CG, N_ATOM_BASIS), jnp.float32),
    )(S0, S1, c_col)


_NC, _NS = 2, 16
_NW = _NC * _NS


def _sc_mesh():
    return plsc.VectorSubcoreMesh(
        core_axis_name="c", subcore_axis_name="s", num_cores=_NC, num_subcores=_NS)


_CH = 40
_G = 32


def _msg_scatter_sc(rn, f_ch, src_ch, dst_ch, zeros, n_nodes, n_chunks):
    N = n_nodes
    BR = zeros.shape[0]
    nbl = N // BR
    kz = -(-nbl // _NS)
    cpw = -(-n_chunks // _NW)
    cpw = -(-cpw // _G) * _G
    ngr = cpw // _G

    cpad = _NW * cpw
    src_p = jnp.zeros((cpad, _CH), jnp.int32).at[:n_chunks].set(src_ch)
    dst_p = jnp.zeros((cpad, _CH), jnp.int32).at[:n_chunks].set(dst_ch)

    def body(rn_hbm, f_hbm, src_hbm, dst_hbm, z_hbm, out_hbm,
             agg_sh, isa_v, ida_v, isb_v, idb_v,
             fa_v, rsa_v, rda_v, fb_v, rsb_v, rdb_v, sema, semb, semi):
        c = lax.axis_index("c")
        s = lax.axis_index("s")
        w = s * _NC + c
        base = w * cpw
        nloc = jnp.minimum(jnp.maximum(n_chunks - base, 0), cpw)

        def zero_blk(k2, carry):
            bid = k2 * _NS + s

            @pl.when(bid < nbl)
            def _():
                off = pl.multiple_of(bid * BR, BR)
                pltpu.sync_copy(z_hbm, agg_sh.at[pl.ds(off, BR)])

            return carry

        def fetch_idx(g, is_v, id_v):
            goff = pl.multiple_of(base + g * _G, _G)
            pltpu.async_copy(src_hbm.at[pl.ds(goff, _G)], is_v, semi)
            pltpu.async_copy(dst_hbm.at[pl.ds(goff, _G)], id_v, semi)

        def drain_idx(is_v, id_v):
            pltpu.make_async_copy(src_hbm.at[pl.ds(0, _G)], is_v, semi).wait()
            pltpu.make_async_copy(src_hbm.at[pl.ds(0, _G)], id_v, semi).wait()

        fetch_idx(0, isa_v, ida_v)
        lax.fori_loop(0, kz, zero_blk, 0)
        plsc.subcore_barrier()

        def fire(g, t, is_v, id_v, f_v, rs_v, rd_v, sem):
            j = g * _G + t

            @pl.when(j < nloc)
            def _():
                foff = pl.multiple_of((base + j) * _CH, _CH)
                pltpu.async_copy(f_hbm.at[pl.ds(foff, _CH)], f_v, sem)
                pltpu.async_copy(rn_hbm.at[is_v.at[t]], rs_v, sem)
                pltpu.async_copy(rn_hbm.at[id_v.at[t]], rd_v, sem)

        def drain_compute(g, t, is_v, id_v, f_v, rs_v, rd_v, sem):
            j = g * _G + t

            @pl.when(j < nloc)
            def _():
                pltpu.make_async_copy(f_hbm.at[pl.ds(0, _CH)], f_v, sem).wait()
                pltpu.make_async_copy(f_hbm.at[pl.ds(0, _CH)], rs_v, sem).wait()
                pltpu.make_async_copy(f_hbm.at[pl.ds(0, _CH)], rd_v, sem).wait()

                @plsc.parallel_loop(0, _CH, unroll=8)
                def _(r):
                    for jj in range(8):
                        sl = pl.ds(jj * 16, 16)
                        fv = f_v[r, sl]
                        rs_v[r, sl] = rs_v[r, sl] * fv
                        rd_v[r, sl] = rd_v[r, sl] * fv

                pltpu.sync_copy(rs_v, agg_sh.at[id_v.at[t]], add=True)
                pltpu.sync_copy(rd_v, agg_sh.at[is_v.at[t]], add=True)

        def group(g, carry):
            geven = g % 2 == 0

            def run(is_v, id_v, isn_v, idn_v):
                drain_idx(is_v, id_v)

                @pl.when(g + 1 < ngr)
                def _():
                    fetch_idx(g + 1, isn_v, idn_v)

                fire(g, 0, is_v, id_v, fa_v, rsa_v, rda_v, sema)

                def pair(t2, carry2):
                    t0 = t2 * 2
                    fire(g, t0 + 1, is_v, id_v, fb_v, rsb_v, rdb_v, semb)
                    drain_compute(g, t0, is_v, id_v, fa_v, rsa_v, rda_v, sema)

                    @pl.when(t0 + 2 < _G)
                    def _():
                        fire(g, t0 + 2, is_v, id_v, fa_v, rsa_v, rda_v, sema)

                    drain_compute(g, t0 + 1, is_v, id_v, fb_v, rsb_v, rdb_v,
                                  semb)
                    return carry2

                lax.fori_loop(0, _G // 2, pair, 0)

            @pl.when(geven)
            def _():
                run(isa_v, ida_v, isb_v, idb_v)

            @pl.when(jnp.logical_not(geven))
            def _():
                run(isb_v, idb_v, isa_v, ida_v)

            return carry

        lax.fori_loop(0, ngr, group, 0)
        plsc.subcore_barrier()

        def read_blk(k2, carry):
            bid = k2 * _NS + s

            @pl.when(bid < nbl)
            def _():
                off = pl.multiple_of(bid * BR, BR)
                pltpu.sync_copy(agg_sh.at[pl.ds(off, BR)],
                                out_hbm.at[c, pl.ds(off, BR)])

            return carry

        lax.fori_loop(0, kz, read_blk, 0)

    return pl.kernel(
        body,
        out_type=jax.ShapeDtypeStruct((_NC, N, N_FILTERS), jnp.float32),
        mesh=_sc_mesh(),
        scratch_types=[
            pltpu.VMEM_SHARED((N, N_FILTERS), jnp.float32),
            pltpu.VMEM((_G, _CH), jnp.int32),
            pltpu.VMEM((_G, _CH), jnp.int32),
            pltpu.VMEM((_G, _CH), jnp.int32),
            pltpu.VMEM((_G, _CH), jnp.int32),
            pltpu.VMEM((_CH, N_FILTERS), jnp.float32),
            pltpu.VMEM((_CH, N_FILTERS), jnp.float32),
            pltpu.VMEM((_CH, N_FILTERS), jnp.float32),
            pltpu.VMEM((_CH, N_FILTERS), jnp.float32),
            pltpu.VMEM((_CH, N_FILTERS), jnp.float32),
            pltpu.VMEM((_CH, N_FILTERS), jnp.float32),
            pltpu.SemaphoreType.DMA,
            pltpu.SemaphoreType.DMA,
            pltpu.SemaphoreType.DMA,
        ],
    )(rn, f_ch, src_p, dst_p, zeros)


def _seg_sum_sc(x, map_ch, zeros, n_rows, n_out):
    CH = 80
    n_chunks = n_rows // CH
    kmax = -(-n_chunks // _NW)
    BR = 40
    nbl = n_out // BR
    kz = -(-nbl // _NS)

    def body(x_hbm, map_hbm, z_hbm, out_hbm, acc_sh, idx_v, x_v, sem):
        c = lax.axis_index("c")
        s = lax.axis_index("s")
        w = s * _NC + c

        def zero_blk(k2, carry):
            bid = k2 * _NS + s

            @pl.when(bid < nbl)
            def _():
                off = pl.multiple_of(bid * BR, BR)
                pltpu.sync_copy(z_hbm, acc_sh.at[pl.ds(off, BR)])

            return carry

        lax.fori_loop(0, kz, zero_blk, 0)
        plsc.subcore_barrier()

        def chunk(k, carry):
            cid = k * _NW + w

            @pl.when(cid < n_chunks)
            def _():
                pltpu.sync_copy(map_hbm.at[cid], idx_v)
                roff = pl.multiple_of(cid * CH, CH)
                pltpu.sync_copy(x_hbm.at[pl.ds(roff, CH)], x_v)
                pltpu.sync_copy(x_v, acc_sh.at[idx_v], add=True)

            return carry

        lax.fori_loop(0, kmax, chunk, 0)
        plsc.subcore_barrier()

        def read_blk(k2, carry):
            bid = k2 * _NS + s

            @pl.when(bid < nbl)
            def _():
                off = pl.multiple_of(bid * BR, BR)
                pltpu.sync_copy(acc_sh.at[pl.ds(off, BR)],
                                out_hbm.at[c, pl.ds(off, BR)])

            return carry

        lax.fori_loop(0, kz, read_blk, 0)

    return pl.kernel(
        body,
        out_type=jax.ShapeDtypeStruct((_NC, n_out, N_FILTERS), jnp.float32),
        mesh=_sc_mesh(),
        scratch_types=[
            pltpu.VMEM_SHARED((n_out, N_FILTERS), jnp.float32),
            pltpu.VMEM((CH,), jnp.int32),
            pltpu.VMEM((CH, N_FILTERS), jnp.float32),
            pltpu.SemaphoreType.DMA,
        ],
    )(x, map_ch, zeros)


def _gather_add_sc(x, table, map_ch, n_rows):
    CH = 80
    n_chunks = n_rows // CH
    kmax = -(-n_chunks // _NW)

    def body(x_hbm, tab_hbm, map_hbm, out_hbm, idx_v, x_v, g_v, sem):
        c = lax.axis_index("c")
        s = lax.axis_index("s")
        w = s * _NC + c

        def chunk(k, carry):
            cid = k * _NW + w

            @pl.when(cid < n_chunks)
            def _():
                pltpu.sync_copy(map_hbm.at[cid], idx_v)
                roff = pl.multiple_of(cid * CH, CH)
                d1 = pltpu.async_copy(tab_hbm.at[idx_v], g_v, sem)
                pltpu.sync_copy(x_hbm.at[pl.ds(roff, CH)], x_v)
                d1.wait()

                def row(r, carry2):
                    for j in range(8):
                        sl = pl.ds(j * 16, 16)
                        x_v[r, sl] = x_v[r, sl] + g_v[r, sl]
                    return carry2

                lax.fori_loop(0, CH, row, 0)
                pltpu.sync_copy(x_v, out_hbm.at[pl.ds(roff, CH)])

            return carry

        lax.fori_loop(0, kmax, chunk, 0)

    return pl.kernel(
        body,
        out_type=jax.ShapeDtypeStruct((n_rows, N_FILTERS), jnp.float32),
        mesh=_sc_mesh(),
        scratch_types=[
            pltpu.VMEM((CH,), jnp.int32),
            pltpu.VMEM((CH, N_FILTERS), jnp.float32),
            pltpu.VMEM((CH, N_FILTERS), jnp.float32),
            pltpu.SemaphoreType.DMA,
        ],
    )(x, table, map_ch)


def kernel(z, xyz, cg_xyz, mapping, nbr_list, CG_nbr_list, embed,
           W_ef1, b_ef1, W_ef2, b_ef2, W_nf, b_nf, W_u1, b_u1, W_u2, b_u2):
    z2d = z.astype(jnp.int32)[:, None]
    m2d = mapping.astype(jnp.int32)[:, None]

    src_af = nbr_list[:, 0].astype(jnp.int32)
    dst_af = nbr_list[:, 1].astype(jnp.int32)
    src_cf = CG_nbr_list[:, 0].astype(jnp.int32)
    dst_cf = CG_nbr_list[:, 1].astype(jnp.int32)
    xyz8 = jnp.zeros((N_ATOMS, 8), jnp.float32).at[:, :3].set(xyz.astype(jnp.float32))
    cg8 = jnp.zeros((N_CG, 8), jnp.float32).at[:, :3].set(cg_xyz.astype(jnp.float32))
    gpair_a = jnp.stack([xyz8[src_af], xyz8[dst_af]])
    gpair_c = jnp.stack([cg8[src_cf], cg8[dst_cf]])

    src_a = src_af.reshape(N_EDGES // _CH, _CH)
    dst_a = dst_af.reshape(N_EDGES // _CH, _CH)
    src_c = src_cf.reshape(N_CG_EDGES // _CH, _CH)
    dst_c = dst_cf.reshape(N_CG_EDGES // _CH, _CH)
    map_ch = mapping.astype(jnp.int32).reshape(N_ATOMS // 80, 80)
    zeros_br = jnp.zeros((40, N_FILTERS), jnp.float32)
    zeros_a = jnp.zeros((1000, N_FILTERS), jnp.float32)
    zeros_c = jnp.zeros((200, N_FILTERS), jnp.float32)

    s_i = _embed(z2d, embed)
    counts = _counts(m2d)
    c_col = counts.T

    S_I = None
    for i in range(N_CONV):
        f = _edge_filter(gpair_a, W_ef1[i], b_ef1[i][None, :], W_ef2[i], b_ef2[i][None, :])
        rn = _rn(s_i, W_nf[i], b_nf[i][None, :])
        agg2 = _msg_scatter_sc(rn, f, src_a, dst_a, zeros_a, N_ATOMS, N_EDGES // _CH)
        s_i = _update(s_i, agg2[0], agg2[1], W_u1[i], b_u1[i][None, :], W_u2[i], b_u2[i][None, :])

        S_parts = _seg_sum_sc(s_i, map_ch, zeros_br, N_ATOMS, N_CG)
        S_input = _div(S_parts[0], S_parts[1], c_col)
        if i == 0:
            S_I = S_input

        j = N_CONV + i
        fc = _edge_filter(gpair_c, W_ef1[j], b_ef1[j][None, :], W_ef2[j], b_ef2[j][None, :])
        Rn = _rn(S_input, W_nf[j], b_nf[j][None, :])
        Agg2 = _msg_scatter_sc(Rn, fc, src_c, dst_c, zeros_c, N_CG, N_CG_EDGES // _CH)
        S_I = _update(S_I, Agg2[0], Agg2[1], W_u1[j], b_u1[j][None, :], W_u2[j], b_u2[j][None, :])

        if i < N_CONV - 1:
            s_i = _gather_add_sc(s_i, S_I, map_ch, N_ATOMS)

    return S_I


# speedup vs baseline: 1.8670x; 1.5592x over previous
import functools
import numpy as np
import jax
import jax.numpy as jnp
from jax import lax
from jax.experimental import pallas as pl
from jax.experimental.pallas import tpu as pltpu
from jax.experimental.pallas import tpu_sc as plsc

N_ATOMS = 10000
N_CG = 1000
N_EDGES = 320000
N_CG_EDGES = 32000
N_ATOM_BASIS = 128
N_FILTERS = 128
N_GAUSSIANS = 50
N_CONV = 3
CUTOFF = 5.0

_OFFSETS = np.linspace(0.0, CUTOFF, N_GAUSSIANS).astype(np.float32)
_WIDTH = float(_OFFSETS[1] - _OFFSETS[0])
_COEFF = -0.5 / _WIDTH**2
_LOG2 = float(np.log(2.0))


def _ssp(x):
    return jnp.logaddexp(x, 0.0) - _LOG2


def _embed_body(z_ref, emb_ref, out_ref):
    z = z_ref[...]
    oh = (z == jax.lax.broadcasted_iota(jnp.int32, (1, 100), 1)).astype(jnp.float32)
    out_ref[...] = jnp.dot(oh, emb_ref[...], preferred_element_type=jnp.float32)


def _embed(z2d, embed):
    n = z2d.shape[0]
    blk = 1000
    return pl.pallas_call(
        _embed_body,
        grid=(n // blk,),
        in_specs=[
            pl.BlockSpec((blk, 1), lambda i: (i, 0)),
            pl.BlockSpec((100, N_ATOM_BASIS), lambda i: (0, 0)),
        ],
        out_specs=pl.BlockSpec((blk, N_ATOM_BASIS), lambda i: (i, 0)),
        out_shape=jax.ShapeDtypeStruct((n, N_ATOM_BASIS), jnp.float32),
    )(z2d, embed)


def _counts_body(m_ref, out_ref):
    i = pl.program_id(0)
    m = m_ref[...]
    oh = (m == jax.lax.broadcasted_iota(jnp.int32, (1, N_CG), 1)).astype(jnp.float32)
    c = jnp.sum(oh, axis=0, keepdims=True)

    @pl.when(i == 0)
    def _():
        out_ref[...] = jnp.zeros_like(out_ref)

    out_ref[...] += c


def _counts(m2d):
    n = m2d.shape[0]
    blk = 1000
    return pl.pallas_call(
        _counts_body,
        grid=(n // blk,),
        in_specs=[pl.BlockSpec((blk, 1), lambda i: (i, 0))],
        out_specs=pl.BlockSpec((1, N_CG), lambda i: (0, 0)),
        out_shape=jax.ShapeDtypeStruct((1, N_CG), jnp.float32),
    )(m2d)


def _edge_filter_body(g_ref, W1_ref, b1_ref, W2_ref, b2_ref, out_ref):
    d = jnp.sqrt(jnp.sum(g_ref[...], axis=1, keepdims=True))
    offs = jax.lax.broadcasted_iota(jnp.int32, (1, N_GAUSSIANS), 1).astype(jnp.float32) * (CUTOFF / (N_GAUSSIANS - 1))
    g = jnp.exp(_COEFF * (d - offs) ** 2)
    h = _ssp(jnp.dot(g, W1_ref[...], preferred_element_type=jnp.float32) + b1_ref[...])
    out_ref[...] = jnp.dot(h, W2_ref[...], preferred_element_type=jnp.float32) + b2_ref[...]


def _edge_filter(diff2, W1, b1, W2, b2):
    e = diff2.shape[0]
    blk = 2000
    return pl.pallas_call(
        _edge_filter_body,
        grid=(e // blk,),
        in_specs=[
            pl.BlockSpec((blk, 16), lambda i: (i, 0)),
            pl.BlockSpec((N_GAUSSIANS, N_GAUSSIANS), lambda i: (0, 0)),
            pl.BlockSpec((1, N_GAUSSIANS), lambda i: (0, 0)),
            pl.BlockSpec((N_GAUSSIANS, N_FILTERS), lambda i: (0, 0)),
            pl.BlockSpec((1, N_FILTERS), lambda i: (0, 0)),
        ],
        out_specs=pl.BlockSpec((blk, N_FILTERS), lambda i: (i, 0)),
        out_shape=jax.ShapeDtypeStruct((e, N_FILTERS), jnp.float32),
    )(diff2, W1, b1, W2, b2)


def _rn_body(s_ref, W_ref, b_ref, out_ref):
    out_ref[...] = jnp.dot(s_ref[...], W_ref[...], preferred_element_type=jnp.float32) + b_ref[...]


def _rn(s, W, b):
    n = s.shape[0]
    blk = min(n, 2000)
    return pl.pallas_call(
        _rn_body,
        grid=(n // blk,),
        in_specs=[
            pl.BlockSpec((blk, N_ATOM_BASIS), lambda i: (i, 0)),
            pl.BlockSpec((N_ATOM_BASIS, N_FILTERS), lambda i: (0, 0)),
            pl.BlockSpec((1, N_FILTERS), lambda i: (0, 0)),
        ],
        out_specs=pl.BlockSpec((blk, N_FILTERS), lambda i: (i, 0)),
        out_shape=jax.ShapeDtypeStruct((n, N_FILTERS), jnp.float32),
    )(s, W, b)


def _update_body(s_ref, a0_ref, a1_ref, W1_ref, b1_ref, W2_ref, b2_ref, out_ref):
    agg = a0_ref[...] + a1_ref[...]
    h = _ssp(jnp.dot(agg, W1_ref[...], preferred_element_type=jnp.float32) + b1_ref[...])
    out_ref[...] = s_ref[...] + jnp.dot(h, W2_ref[...], preferred_element_type=jnp.float32) + b2_ref[...]


def _update(s, a0, a1, Wu1, bu1, Wu2, bu2):
    n = s.shape[0]
    blk = min(n, 2000)
    return pl.pallas_call(
        _update_body,
        grid=(n // blk,),
        in_specs=[
            pl.BlockSpec((blk, N_ATOM_BASIS), lambda i: (i, 0)),
            pl.BlockSpec((blk, N_FILTERS), lambda i: (i, 0)),
            pl.BlockSpec((blk, N_FILTERS), lambda i: (i, 0)),
            pl.BlockSpec((N_FILTERS, N_ATOM_BASIS), lambda i: (0, 0)),
            pl.BlockSpec((1, N_ATOM_BASIS), lambda i: (0, 0)),
            pl.BlockSpec((N_ATOM_BASIS, N_ATOM_BASIS), lambda i: (0, 0)),
            pl.BlockSpec((1, N_ATOM_BASIS), lambda i: (0, 0)),
        ],
        out_specs=pl.BlockSpec((blk, N_ATOM_BASIS), lambda i: (i, 0)),
        out_shape=jax.ShapeDtypeStruct((n, N_ATOM_BASIS), jnp.float32),
    )(s, a0, a1, Wu1, bu1, Wu2, bu2)


def _div_body(S0_ref, S1_ref, c_ref, out_ref):
    c = jnp.maximum(c_ref[...], 1.0)
    out_ref[...] = (S0_ref[...] + S1_ref[...]) / c


def _div(S0, S1, c_col):
    return pl.pallas_call(
        _div_body,
        grid=(1,),
        in_specs=[
            pl.BlockSpec((N_CG, N_ATOM_BASIS), lambda i: (0, 0)),
            pl.BlockSpec((N_CG, N_ATOM_BASIS), lambda i: (0, 0)),
            pl.BlockSpec((N_CG, 1), lambda i: (0, 0)),
        ],
        out_specs=pl.BlockSpec((N_CG, N_ATOM_BASIS), lambda i: (0, 0)),
        out_shape=jax.ShapeDtypeStruct((N_CG, N_ATOM_BASIS), jnp.float32),
    )(S0, S1, c_col)


_NC, _NS = 2, 16
_NW = _NC * _NS


def _sc_mesh():
    return plsc.VectorSubcoreMesh(
        core_axis_name="c", subcore_axis_name="s", num_cores=_NC, num_subcores=_NS)


_CH = 40
_G = 32


def _msg_scatter_sc(rn, f_ch, src_ch, dst_ch, zeros, n_nodes, n_chunks):
    N = n_nodes
    BR = zeros.shape[0]
    nbl = N // BR
    kz = -(-nbl // _NS)
    cpw = -(-n_chunks // _NW)
    cpw = -(-cpw // _G) * _G
    ngr = cpw // _G

    cpad = _NW * cpw
    src_p = jnp.zeros((cpad, _CH), jnp.int32).at[:n_chunks].set(src_ch)
    dst_p = jnp.zeros((cpad, _CH), jnp.int32).at[:n_chunks].set(dst_ch)

    def body(rn_hbm, f_hbm, src_hbm, dst_hbm, z_hbm, out_hbm,
             agg_sh, isa_v, ida_v, isb_v, idb_v,
             fa_v, rsa_v, rda_v, fb_v, rsb_v, rdb_v, sema, semb, semi):
        c = lax.axis_index("c")
        s = lax.axis_index("s")
        w = s * _NC + c
        base = w * cpw
        nloc = jnp.minimum(jnp.maximum(n_chunks - base, 0), cpw)

        def zero_blk(k2, carry):
            bid = k2 * _NS + s

            @pl.when(bid < nbl)
            def _():
                off = pl.multiple_of(bid * BR, BR)
                pltpu.sync_copy(z_hbm, agg_sh.at[pl.ds(off, BR)])

            return carry

        def fetch_idx(g, is_v, id_v):
            goff = pl.multiple_of(base + g * _G, _G)
            pltpu.async_copy(src_hbm.at[pl.ds(goff, _G)], is_v, semi)
            pltpu.async_copy(dst_hbm.at[pl.ds(goff, _G)], id_v, semi)

        def drain_idx(is_v, id_v):
            pltpu.make_async_copy(src_hbm.at[pl.ds(0, _G)], is_v, semi).wait()
            pltpu.make_async_copy(src_hbm.at[pl.ds(0, _G)], id_v, semi).wait()

        fetch_idx(0, isa_v, ida_v)
        lax.fori_loop(0, kz, zero_blk, 0)
        plsc.subcore_barrier()

        def fire(g, t, is_v, id_v, f_v, rs_v, rd_v, sem):
            j = g * _G + t

            @pl.when(j < nloc)
            def _():
                foff = pl.multiple_of((base + j) * _CH, _CH)
                pltpu.async_copy(f_hbm.at[pl.ds(foff, _CH)], f_v, sem)
                pltpu.async_copy(rn_hbm.at[is_v.at[t]], rs_v, sem)
                pltpu.async_copy(rn_hbm.at[id_v.at[t]], rd_v, sem)

        def drain_compute(g, t, is_v, id_v, f_v, rs_v, rd_v, sem):
            j = g * _G + t

            @pl.when(j < nloc)
            def _():
                pltpu.make_async_copy(f_hbm.at[pl.ds(0, _CH)], f_v, sem).wait()
                pltpu.make_async_copy(f_hbm.at[pl.ds(0, _CH)], rs_v, sem).wait()
                pltpu.make_async_copy(f_hbm.at[pl.ds(0, _CH)], rd_v, sem).wait()

                @plsc.parallel_loop(0, _CH, unroll=8)
                def _(r):
                    for jj in range(8):
                        sl = pl.ds(jj * 16, 16)
                        fv = f_v[r, sl]
                        rs_v[r, sl] = rs_v[r, sl] * fv
                        rd_v[r, sl] = rd_v[r, sl] * fv

                pltpu.sync_copy(rs_v, agg_sh.at[id_v.at[t]], add=True)
                pltpu.sync_copy(rd_v, agg_sh.at[is_v.at[t]], add=True)

        def group(g, carry):
            geven = g % 2 == 0

            def run(is_v, id_v, isn_v, idn_v):
                drain_idx(is_v, id_v)

                @pl.when(g + 1 < ngr)
                def _():
                    fetch_idx(g + 1, isn_v, idn_v)

                fire(g, 0, is_v, id_v, fa_v, rsa_v, rda_v, sema)

                def pair(t2, carry2):
                    t0 = t2 * 2
                    fire(g, t0 + 1, is_v, id_v, fb_v, rsb_v, rdb_v, semb)
                    drain_compute(g, t0, is_v, id_v, fa_v, rsa_v, rda_v, sema)

                    @pl.when(t0 + 2 < _G)
                    def _():
                        fire(g, t0 + 2, is_v, id_v, fa_v, rsa_v, rda_v, sema)

                    drain_compute(g, t0 + 1, is_v, id_v, fb_v, rsb_v, rdb_v,
                                  semb)
                    return carry2

                lax.fori_loop(0, _G // 2, pair, 0)

            @pl.when(geven)
            def _():
                run(isa_v, ida_v, isb_v, idb_v)

            @pl.when(jnp.logical_not(geven))
            def _():
                run(isb_v, idb_v, isa_v, ida_v)

            return carry

        lax.fori_loop(0, ngr, group, 0)
        plsc.subcore_barrier()

        def read_blk(k2, carry):
            bid = k2 * _NS + s

            @pl.when(bid < nbl)
            def _():
                off = pl.multiple_of(bid * BR, BR)
                pltpu.sync_copy(agg_sh.at[pl.ds(off, BR)],
                                out_hbm.at[c, pl.ds(off, BR)])

            return carry

        lax.fori_loop(0, kz, read_blk, 0)

    return pl.kernel(
        body,
        out_type=jax.ShapeDtypeStruct((_NC, N, N_FILTERS), jnp.float32),
        mesh=_sc_mesh(),
        scratch_types=[
            pltpu.VMEM_SHARED((N, N_FILTERS), jnp.float32),
            pltpu.VMEM((_G, _CH), jnp.int32),
            pltpu.VMEM((_G, _CH), jnp.int32),
            pltpu.VMEM((_G, _CH), jnp.int32),
            pltpu.VMEM((_G, _CH), jnp.int32),
            pltpu.VMEM((_CH, N_FILTERS), jnp.float32),
            pltpu.VMEM((_CH, N_FILTERS), jnp.float32),
            pltpu.VMEM((_CH, N_FILTERS), jnp.float32),
            pltpu.VMEM((_CH, N_FILTERS), jnp.float32),
            pltpu.VMEM((_CH, N_FILTERS), jnp.float32),
            pltpu.VMEM((_CH, N_FILTERS), jnp.float32),
            pltpu.SemaphoreType.DMA,
            pltpu.SemaphoreType.DMA,
            pltpu.SemaphoreType.DMA,
        ],
    )(rn, f_ch, src_p, dst_p, zeros)


def _seg_sum_sc(x, map_ch, zeros, n_rows, n_out):
    CH = 80
    n_chunks = n_rows // CH
    kmax = -(-n_chunks // _NW)
    BR = 40
    nbl = n_out // BR
    kz = -(-nbl // _NS)

    def body(x_hbm, map_hbm, z_hbm, out_hbm, acc_sh, idx_v, x_v, sem):
        c = lax.axis_index("c")
        s = lax.axis_index("s")
        w = s * _NC + c

        def zero_blk(k2, carry):
            bid = k2 * _NS + s

            @pl.when(bid < nbl)
            def _():
                off = pl.multiple_of(bid * BR, BR)
                pltpu.sync_copy(z_hbm, acc_sh.at[pl.ds(off, BR)])

            return carry

        lax.fori_loop(0, kz, zero_blk, 0)
        plsc.subcore_barrier()

        def chunk(k, carry):
            cid = k * _NW + w

            @pl.when(cid < n_chunks)
            def _():
                pltpu.sync_copy(map_hbm.at[cid], idx_v)
                roff = pl.multiple_of(cid * CH, CH)
                pltpu.sync_copy(x_hbm.at[pl.ds(roff, CH)], x_v)
                pltpu.sync_copy(x_v, acc_sh.at[idx_v], add=True)

            return carry

        lax.fori_loop(0, kmax, chunk, 0)
        plsc.subcore_barrier()

        def read_blk(k2, carry):
            bid = k2 * _NS + s

            @pl.when(bid < nbl)
            def _():
                off = pl.multiple_of(bid * BR, BR)
                pltpu.sync_copy(acc_sh.at[pl.ds(off, BR)],
                                out_hbm.at[c, pl.ds(off, BR)])

            return carry

        lax.fori_loop(0, kz, read_blk, 0)

    return pl.kernel(
        body,
        out_type=jax.ShapeDtypeStruct((_NC, n_out, N_FILTERS), jnp.float32),
        mesh=_sc_mesh(),
        scratch_types=[
            pltpu.VMEM_SHARED((n_out, N_FILTERS), jnp.float32),
            pltpu.VMEM((CH,), jnp.int32),
            pltpu.VMEM((CH, N_FILTERS), jnp.float32),
            pltpu.SemaphoreType.DMA,
        ],
    )(x, map_ch, zeros)


def _edge_diff_sc(xyz16, src_ch, dst_ch):
    n_chunks = src_ch.shape[0]
    kmax = -(-n_chunks // _NW)

    def body(xyz_hbm, src_hbm, dst_hbm, out_hbm, is_v, id_v, a_v, b_v, sem):
        c = lax.axis_index("c")
        s = lax.axis_index("s")
        w = s * _NC + c

        def chunk(k, carry):
            cid = k * _NW + w

            @pl.when(cid < n_chunks)
            def _():
                pltpu.sync_copy(src_hbm.at[cid], is_v)
                pltpu.sync_copy(dst_hbm.at[cid], id_v)
                d1 = pltpu.async_copy(xyz_hbm.at[is_v], a_v, sem)
                d2 = pltpu.async_copy(xyz_hbm.at[id_v], b_v, sem)
                d1.wait()
                d2.wait()

                @plsc.parallel_loop(0, _CH, unroll=8)
                def _(r):
                    d = a_v[r, :] - b_v[r, :]
                    a_v[r, :] = d * d

                roff = pl.multiple_of(cid * _CH, _CH)
                pltpu.sync_copy(a_v, out_hbm.at[pl.ds(roff, _CH)])

            return carry

        lax.fori_loop(0, kmax, chunk, 0)

    return pl.kernel(
        body,
        out_type=jax.ShapeDtypeStruct((n_chunks * _CH, 16), jnp.float32),
        mesh=_sc_mesh(),
        compiler_params=pltpu.CompilerParams(use_tc_tiling_on_sc=False),
        scratch_types=[
            pltpu.VMEM((_CH,), jnp.int32),
            pltpu.VMEM((_CH,), jnp.int32),
            pltpu.VMEM((_CH, 16), jnp.float32),
            pltpu.VMEM((_CH, 16), jnp.float32),
            pltpu.SemaphoreType.DMA,
        ],
    )(xyz16, src_ch, dst_ch)


def _gather_add_sc(x, table, map_ch, n_rows):
    CH = 80
    n_chunks = n_rows // CH
    kmax = -(-n_chunks // _NW)

    def body(x_hbm, tab_hbm, map_hbm, out_hbm, idx_v, x_v, g_v, sem):
        c = lax.axis_index("c")
        s = lax.axis_index("s")
        w = s * _NC + c

        def chunk(k, carry):
            cid = k * _NW + w

            @pl.when(cid < n_chunks)
            def _():
                pltpu.sync_copy(map_hbm.at[cid], idx_v)
                roff = pl.multiple_of(cid * CH, CH)
                d1 = pltpu.async_copy(tab_hbm.at[idx_v], g_v, sem)
                pltpu.sync_copy(x_hbm.at[pl.ds(roff, CH)], x_v)
                d1.wait()

                def row(r, carry2):
                    for j in range(8):
                        sl = pl.ds(j * 16, 16)
                        x_v[r, sl] = x_v[r, sl] + g_v[r, sl]
                    return carry2

                lax.fori_loop(0, CH, row, 0)
                pltpu.sync_copy(x_v, out_hbm.at[pl.ds(roff, CH)])

            return carry

        lax.fori_loop(0, kmax, chunk, 0)

    return pl.kernel(
        body,
        out_type=jax.ShapeDtypeStruct((n_rows, N_FILTERS), jnp.float32),
        mesh=_sc_mesh(),
        scratch_types=[
            pltpu.VMEM((CH,), jnp.int32),
            pltpu.VMEM((CH, N_FILTERS), jnp.float32),
            pltpu.VMEM((CH, N_FILTERS), jnp.float32),
            pltpu.SemaphoreType.DMA,
        ],
    )(x, table, map_ch)


def kernel(z, xyz, cg_xyz, mapping, nbr_list, CG_nbr_list, embed,
           W_ef1, b_ef1, W_ef2, b_ef2, W_nf, b_nf, W_u1, b_u1, W_u2, b_u2):
    z2d = z.astype(jnp.int32)[:, None]
    m2d = mapping.astype(jnp.int32)[:, None]

    src_a = nbr_list[:, 0].astype(jnp.int32).reshape(N_EDGES // _CH, _CH)
    dst_a = nbr_list[:, 1].astype(jnp.int32).reshape(N_EDGES // _CH, _CH)
    src_c = CG_nbr_list[:, 0].astype(jnp.int32).reshape(N_CG_EDGES // _CH, _CH)
    dst_c = CG_nbr_list[:, 1].astype(jnp.int32).reshape(N_CG_EDGES // _CH, _CH)
    xyz16 = jnp.zeros((N_ATOMS, 16), jnp.float32).at[:, :3].set(xyz.astype(jnp.float32))
    cg16 = jnp.zeros((N_CG, 16), jnp.float32).at[:, :3].set(cg_xyz.astype(jnp.float32))
    diff2_a = _edge_diff_sc(xyz16, src_a, dst_a)
    diff2_c = _edge_diff_sc(cg16, src_c, dst_c)
    map_ch = mapping.astype(jnp.int32).reshape(N_ATOMS // 80, 80)
    zeros_br = jnp.zeros((40, N_FILTERS), jnp.float32)
    zeros_a = jnp.zeros((1000, N_FILTERS), jnp.float32)
    zeros_c = jnp.zeros((200, N_FILTERS), jnp.float32)

    s_i = _embed(z2d, embed)
    counts = _counts(m2d)
    c_col = counts.T

    S_I = None
    for i in range(N_CONV):
        f = _edge_filter(diff2_a, W_ef1[i], b_ef1[i][None, :], W_ef2[i], b_ef2[i][None, :])
        rn = _rn(s_i, W_nf[i], b_nf[i][None, :])
        agg2 = _msg_scatter_sc(rn, f, src_a, dst_a, zeros_a, N_ATOMS, N_EDGES // _CH)
        s_i = _update(s_i, agg2[0], agg2[1], W_u1[i], b_u1[i][None, :], W_u2[i], b_u2[i][None, :])

        S_parts = _seg_sum_sc(s_i, map_ch, zeros_br, N_ATOMS, N_CG)
        S_input = _div(S_parts[0], S_parts[1], c_col)
        if i == 0:
            S_I = S_input

        j = N_CONV + i
        fc = _edge_filter(diff2_c, W_ef1[j], b_ef1[j][None, :], W_ef2[j], b_ef2[j][None, :])
        Rn = _rn(S_input, W_nf[j], b_nf[j][None, :])
        Agg2 = _msg_scatter_sc(Rn, fc, src_c, dst_c, zeros_c, N_CG, N_CG_EDGES // _CH)
        S_I = _update(S_I, Agg2[0], Agg2[1], W_u1[j], b_u1[j][None, :], W_u2[j], b_u2[j][None, :])

        if i < N_CONV - 1:
            s_i = _gather_add_sc(s_i, S_I, map_ch, N_ATOMS)

    return S_I


# speedup vs baseline: 1.8672x; 1.0001x over previous
import functools
import numpy as np
import jax
import jax.numpy as jnp
from jax import lax
from jax.experimental import pallas as pl
from jax.experimental.pallas import tpu as pltpu
from jax.experimental.pallas import tpu_sc as plsc

N_ATOMS = 10000
N_CG = 1000
N_EDGES = 320000
N_CG_EDGES = 32000
N_ATOM_BASIS = 128
N_FILTERS = 128
N_GAUSSIANS = 50
N_CONV = 3
CUTOFF = 5.0

_OFFSETS = np.linspace(0.0, CUTOFF, N_GAUSSIANS).astype(np.float32)
_WIDTH = float(_OFFSETS[1] - _OFFSETS[0])
_COEFF = -0.5 / _WIDTH**2
_LOG2 = float(np.log(2.0))


def _ssp(x):
    return jnp.logaddexp(x, 0.0) - _LOG2


def _embed_body(z_ref, emb_ref, out_ref):
    z = z_ref[...]
    oh = (z == jax.lax.broadcasted_iota(jnp.int32, (1, 100), 1)).astype(jnp.float32)
    out_ref[...] = jnp.dot(oh, emb_ref[...], preferred_element_type=jnp.float32)


def _embed(z2d, embed):
    n = z2d.shape[0]
    blk = 1000
    return pl.pallas_call(
        _embed_body,
        grid=(n // blk,),
        in_specs=[
            pl.BlockSpec((blk, 1), lambda i: (i, 0)),
            pl.BlockSpec((100, N_ATOM_BASIS), lambda i: (0, 0)),
        ],
        out_specs=pl.BlockSpec((blk, N_ATOM_BASIS), lambda i: (i, 0)),
        out_shape=jax.ShapeDtypeStruct((n, N_ATOM_BASIS), jnp.float32),
    )(z2d, embed)


def _counts_body(m_ref, out_ref):
    i = pl.program_id(0)
    m = m_ref[...]
    oh = (m == jax.lax.broadcasted_iota(jnp.int32, (1, N_CG), 1)).astype(jnp.float32)
    c = jnp.sum(oh, axis=0, keepdims=True)

    @pl.when(i == 0)
    def _():
        out_ref[...] = jnp.zeros_like(out_ref)

    out_ref[...] += c


def _counts(m2d):
    n = m2d.shape[0]
    blk = 1000
    return pl.pallas_call(
        _counts_body,
        grid=(n // blk,),
        in_specs=[pl.BlockSpec((blk, 1), lambda i: (i, 0))],
        out_specs=pl.BlockSpec((1, N_CG), lambda i: (0, 0)),
        out_shape=jax.ShapeDtypeStruct((1, N_CG), jnp.float32),
    )(m2d)


def _edge_filter_body(g_ref, W1_ref, b1_ref, W2_ref, b2_ref, out_ref):
    d = jnp.sqrt(jnp.sum(g_ref[...], axis=1, keepdims=True))
    offs = jax.lax.broadcasted_iota(jnp.int32, (1, N_GAUSSIANS), 1).astype(jnp.float32) * (CUTOFF / (N_GAUSSIANS - 1))
    g = jnp.exp(_COEFF * (d - offs) ** 2)
    h = _ssp(jnp.dot(g, W1_ref[...], preferred_element_type=jnp.float32) + b1_ref[...])
    out_ref[...] = jnp.dot(h, W2_ref[...], preferred_element_type=jnp.float32) + b2_ref[...]


def _edge_filter(diff2, W1, b1, W2, b2):
    e = diff2.shape[0]
    blk = 2000
    return pl.pallas_call(
        _edge_filter_body,
        grid=(e // blk,),
        in_specs=[
            pl.BlockSpec((blk, 16), lambda i: (i, 0)),
            pl.BlockSpec((N_GAUSSIANS, N_GAUSSIANS), lambda i: (0, 0)),
            pl.BlockSpec((1, N_GAUSSIANS), lambda i: (0, 0)),
            pl.BlockSpec((N_GAUSSIANS, N_FILTERS), lambda i: (0, 0)),
            pl.BlockSpec((1, N_FILTERS), lambda i: (0, 0)),
        ],
        out_specs=pl.BlockSpec((blk, N_FILTERS), lambda i: (i, 0)),
        out_shape=jax.ShapeDtypeStruct((e, N_FILTERS), jnp.float32),
    )(diff2, W1, b1, W2, b2)


def _rn_body(s_ref, W_ref, b_ref, out_ref):
    out_ref[...] = jnp.dot(s_ref[...], W_ref[...], preferred_element_type=jnp.float32) + b_ref[...]


def _rn(s, W, b):
    n = s.shape[0]
    blk = min(n, 2000)
    return pl.pallas_call(
        _rn_body,
        grid=(n // blk,),
        in_specs=[
            pl.BlockSpec((blk, N_ATOM_BASIS), lambda i: (i, 0)),
            pl.BlockSpec((N_ATOM_BASIS, N_FILTERS), lambda i: (0, 0)),
            pl.BlockSpec((1, N_FILTERS), lambda i: (0, 0)),
        ],
        out_specs=pl.BlockSpec((blk, N_FILTERS), lambda i: (i, 0)),
        out_shape=jax.ShapeDtypeStruct((n, N_FILTERS), jnp.float32),
    )(s, W, b)


def _update_body(s_ref, a0_ref, a1_ref, W1_ref, b1_ref, W2_ref, b2_ref, out_ref):
    agg = a0_ref[...] + a1_ref[...]
    h = _ssp(jnp.dot(agg, W1_ref[...], preferred_element_type=jnp.float32) + b1_ref[...])
    out_ref[...] = s_ref[...] + jnp.dot(h, W2_ref[...], preferred_element_type=jnp.float32) + b2_ref[...]


def _update(s, a0, a1, Wu1, bu1, Wu2, bu2):
    n = s.shape[0]
    blk = min(n, 2000)
    return pl.pallas_call(
        _update_body,
        grid=(n // blk,),
        in_specs=[
            pl.BlockSpec((blk, N_ATOM_BASIS), lambda i: (i, 0)),
            pl.BlockSpec((blk, N_FILTERS), lambda i: (i, 0)),
            pl.BlockSpec((blk, N_FILTERS), lambda i: (i, 0)),
            pl.BlockSpec((N_FILTERS, N_ATOM_BASIS), lambda i: (0, 0)),
            pl.BlockSpec((1, N_ATOM_BASIS), lambda i: (0, 0)),
            pl.BlockSpec((N_ATOM_BASIS, N_ATOM_BASIS), lambda i: (0, 0)),
            pl.BlockSpec((1, N_ATOM_BASIS), lambda i: (0, 0)),
        ],
        out_specs=pl.BlockSpec((blk, N_ATOM_BASIS), lambda i: (i, 0)),
        out_shape=jax.ShapeDtypeStruct((n, N_ATOM_BASIS), jnp.float32),
    )(s, a0, a1, Wu1, bu1, Wu2, bu2)


def _div_body(S0_ref, S1_ref, c_ref, out_ref):
    c = jnp.maximum(c_ref[...], 1.0)
    out_ref[...] = (S0_ref[...] + S1_ref[...]) / c


def _div(S0, S1, c_col):
    return pl.pallas_call(
        _div_body,
        grid=(1,),
        in_specs=[
            pl.BlockSpec((N_CG, N_ATOM_BASIS), lambda i: (0, 0)),
            pl.BlockSpec((N_CG, N_ATOM_BASIS), lambda i: (0, 0)),
            pl.BlockSpec((N_CG, 1), lambda i: (0, 0)),
        ],
        out_specs=pl.BlockSpec((N_CG, N_ATOM_BASIS), lambda i: (0, 0)),
        out_shape=jax.ShapeDtypeStruct((N_CG, N_ATOM_BASIS), jnp.float32),
    )(S0, S1, c_col)


_NC, _NS = 2, 16
_NW = _NC * _NS


def _sc_mesh():
    return plsc.VectorSubcoreMesh(
        core_axis_name="c", subcore_axis_name="s", num_cores=_NC, num_subcores=_NS)


_CH = 40
_G = 32


def _msg_scatter_sc(rn, f_ch, src_ch, dst_ch, zeros, n_nodes, n_chunks):
    N = n_nodes
    BR = zeros.shape[0]
    nbl = N // BR
    kz = -(-nbl // _NS)
    cpw = -(-n_chunks // _NW)
    cpw = -(-cpw // _G) * _G
    ngr = cpw // _G

    cpad = _NW * cpw
    src_p = jnp.zeros((cpad, _CH), jnp.int32).at[:n_chunks].set(src_ch)
    dst_p = jnp.zeros((cpad, _CH), jnp.int32).at[:n_chunks].set(dst_ch)

    def body(rn_hbm, f_hbm, src_hbm, dst_hbm, z_hbm, out_hbm,
             agg_sh, isa_v, ida_v, isb_v, idb_v,
             fa_v, rsa_v, rda_v, fb_v, rsb_v, rdb_v, sema, semb, semi):
        c = lax.axis_index("c")
        s = lax.axis_index("s")
        w = s * _NC + c
        base = w * cpw
        nloc = jnp.minimum(jnp.maximum(n_chunks - base, 0), cpw)

        def zero_blk(k2, carry):
            bid = k2 * _NS + s

            @pl.when(bid < nbl)
            def _():
                off = pl.multiple_of(bid * BR, BR)
                pltpu.sync_copy(z_hbm, agg_sh.at[pl.ds(off, BR)])

            return carry

        def fetch_idx(g, is_v, id_v):
            goff = pl.multiple_of(base + g * _G, _G)
            pltpu.async_copy(src_hbm.at[pl.ds(goff, _G)], is_v, semi)
            pltpu.async_copy(dst_hbm.at[pl.ds(goff, _G)], id_v, semi)

        def drain_idx(is_v, id_v):
            pltpu.make_async_copy(src_hbm.at[pl.ds(0, _G)], is_v, semi).wait()
            pltpu.make_async_copy(src_hbm.at[pl.ds(0, _G)], id_v, semi).wait()

        fetch_idx(0, isa_v, ida_v)
        lax.fori_loop(0, kz, zero_blk, 0)
        plsc.subcore_barrier()

        def fire(g, t, is_v, id_v, f_v, rs_v, rd_v, sem):
            j = g * _G + t

            @pl.when(j < nloc)
            def _():
                foff = pl.multiple_of((base + j) * _CH, _CH)
                pltpu.async_copy(f_hbm.at[pl.ds(foff, _CH)], f_v, sem)
                pltpu.async_copy(rn_hbm.at[is_v.at[t]], rs_v, sem)
                pltpu.async_copy(rn_hbm.at[id_v.at[t]], rd_v, sem)

        def drain_compute(g, t, is_v, id_v, f_v, rs_v, rd_v, sem):
            j = g * _G + t

            @pl.when(j < nloc)
            def _():
                pltpu.make_async_copy(f_hbm.at[pl.ds(0, _CH)], f_v, sem).wait()
                pltpu.make_async_copy(f_hbm.at[pl.ds(0, _CH)], rs_v, sem).wait()
                pltpu.make_async_copy(f_hbm.at[pl.ds(0, _CH)], rd_v, sem).wait()

                @plsc.parallel_loop(0, _CH, unroll=8)
                def _(r):
                    for jj in range(8):
                        sl = pl.ds(jj * 16, 16)
                        fv = f_v[r, sl]
                        rs_v[r, sl] = rs_v[r, sl] * fv
                        rd_v[r, sl] = rd_v[r, sl] * fv

                pltpu.sync_copy(rs_v, agg_sh.at[id_v.at[t]], add=True)
                pltpu.sync_copy(rd_v, agg_sh.at[is_v.at[t]], add=True)

        def group(g, carry):
            geven = g % 2 == 0

            def run(is_v, id_v, isn_v, idn_v):
                drain_idx(is_v, id_v)

                @pl.when(g + 1 < ngr)
                def _():
                    fetch_idx(g + 1, isn_v, idn_v)

                fire(g, 0, is_v, id_v, fa_v, rsa_v, rda_v, sema)

                def pair(t2, carry2):
                    t0 = t2 * 2
                    fire(g, t0 + 1, is_v, id_v, fb_v, rsb_v, rdb_v, semb)
                    drain_compute(g, t0, is_v, id_v, fa_v, rsa_v, rda_v, sema)

                    @pl.when(t0 + 2 < _G)
                    def _():
                        fire(g, t0 + 2, is_v, id_v, fa_v, rsa_v, rda_v, sema)

                    drain_compute(g, t0 + 1, is_v, id_v, fb_v, rsb_v, rdb_v,
                                  semb)
                    return carry2

                lax.fori_loop(0, _G // 2, pair, 0)

            @pl.when(geven)
            def _():
                run(isa_v, ida_v, isb_v, idb_v)

            @pl.when(jnp.logical_not(geven))
            def _():
                run(isb_v, idb_v, isa_v, ida_v)

            return carry

        lax.fori_loop(0, ngr, group, 0)
        plsc.subcore_barrier()

        def read_blk(k2, carry):
            bid = k2 * _NS + s

            @pl.when(bid < nbl)
            def _():
                off = pl.multiple_of(bid * BR, BR)
                pltpu.sync_copy(agg_sh.at[pl.ds(off, BR)],
                                out_hbm.at[c, pl.ds(off, BR)])

            return carry

        lax.fori_loop(0, kz, read_blk, 0)

    return pl.kernel(
        body,
        out_type=jax.ShapeDtypeStruct((_NC, N, N_FILTERS), jnp.float32),
        mesh=_sc_mesh(),
        scratch_types=[
            pltpu.VMEM_SHARED((N, N_FILTERS), jnp.float32),
            pltpu.VMEM((_G, _CH), jnp.int32),
            pltpu.VMEM((_G, _CH), jnp.int32),
            pltpu.VMEM((_G, _CH), jnp.int32),
            pltpu.VMEM((_G, _CH), jnp.int32),
            pltpu.VMEM((_CH, N_FILTERS), jnp.float32),
            pltpu.VMEM((_CH, N_FILTERS), jnp.float32),
            pltpu.VMEM((_CH, N_FILTERS), jnp.float32),
            pltpu.VMEM((_CH, N_FILTERS), jnp.float32),
            pltpu.VMEM((_CH, N_FILTERS), jnp.float32),
            pltpu.VMEM((_CH, N_FILTERS), jnp.float32),
            pltpu.SemaphoreType.DMA,
            pltpu.SemaphoreType.DMA,
            pltpu.SemaphoreType.DMA,
        ],
    )(rn, f_ch, src_p, dst_p, zeros)


def _seg_sum_sc(x, map_ch, zeros, n_rows, n_out):
    CH = 80
    n_chunks = n_rows // CH
    kmax = -(-n_chunks // _NW)
    BR = 40
    nbl = n_out // BR
    kz = -(-nbl // _NS)

    def body(x_hbm, map_hbm, z_hbm, out_hbm, acc_sh, idx_v, x_v, sem):
        c = lax.axis_index("c")
        s = lax.axis_index("s")
        w = s * _NC + c

        def zero_blk(k2, carry):
            bid = k2 * _NS + s

            @pl.when(bid < nbl)
            def _():
                off = pl.multiple_of(bid * BR, BR)
                pltpu.sync_copy(z_hbm, acc_sh.at[pl.ds(off, BR)])

            return carry

        lax.fori_loop(0, kz, zero_blk, 0)
        plsc.subcore_barrier()

        def chunk(k, carry):
            cid = k * _NW + w

            @pl.when(cid < n_chunks)
            def _():
                pltpu.sync_copy(map_hbm.at[cid], idx_v)
                roff = pl.multiple_of(cid * CH, CH)
                pltpu.sync_copy(x_hbm.at[pl.ds(roff, CH)], x_v)
                pltpu.sync_copy(x_v, acc_sh.at[idx_v], add=True)

            return carry

        lax.fori_loop(0, kmax, chunk, 0)
        plsc.subcore_barrier()

        def read_blk(k2, carry):
            bid = k2 * _NS + s

            @pl.when(bid < nbl)
            def _():
                off = pl.multiple_of(bid * BR, BR)
                pltpu.sync_copy(acc_sh.at[pl.ds(off, BR)],
                                out_hbm.at[c, pl.ds(off, BR)])

            return carry

        lax.fori_loop(0, kz, read_blk, 0)

    return pl.kernel(
        body,
        out_type=jax.ShapeDtypeStruct((_NC, n_out, N_FILTERS), jnp.float32),
        mesh=_sc_mesh(),
        scratch_types=[
            pltpu.VMEM_SHARED((n_out, N_FILTERS), jnp.float32),
            pltpu.VMEM((CH,), jnp.int32),
            pltpu.VMEM((CH, N_FILTERS), jnp.float32),
            pltpu.SemaphoreType.DMA,
        ],
    )(x, map_ch, zeros)


def _edge_diff_sc(xyz16, src_ch, dst_ch):
    n_chunks = src_ch.shape[0]
    kmax = -(-n_chunks // _NW)

    def body(xyz_hbm, src_hbm, dst_hbm, out_hbm, is_v, id_v, a_v, b_v, sem):
        c = lax.axis_index("c")
        s = lax.axis_index("s")
        w = s * _NC + c

        def chunk(k, carry):
            cid = k * _NW + w

            @pl.when(cid < n_chunks)
            def _():
                pltpu.sync_copy(src_hbm.at[cid], is_v)
                pltpu.sync_copy(dst_hbm.at[cid], id_v)
                d1 = pltpu.async_copy(xyz_hbm.at[is_v], a_v, sem)
                d2 = pltpu.async_copy(xyz_hbm.at[id_v], b_v, sem)
                d1.wait()
                d2.wait()

                @plsc.parallel_loop(0, _CH, unroll=8)
                def _(r):
                    d = a_v[r, :] - b_v[r, :]
                    a_v[r, :] = d * d

                roff = pl.multiple_of(cid * _CH, _CH)
                pltpu.sync_copy(a_v, out_hbm.at[pl.ds(roff, _CH)])

            return carry

        lax.fori_loop(0, kmax, chunk, 0)

    return pl.kernel(
        body,
        out_type=jax.ShapeDtypeStruct((n_chunks * _CH, 16), jnp.float32),
        mesh=_sc_mesh(),
        compiler_params=pltpu.CompilerParams(use_tc_tiling_on_sc=False),
        scratch_types=[
            pltpu.VMEM((_CH,), jnp.int32),
            pltpu.VMEM((_CH,), jnp.int32),
            pltpu.VMEM((_CH, 16), jnp.float32),
            pltpu.VMEM((_CH, 16), jnp.float32),
            pltpu.SemaphoreType.DMA,
        ],
    )(xyz16, src_ch, dst_ch)


def _gather_add_sc(x, table, map_ch, n_rows):
    CH = 80
    n_chunks = n_rows // CH
    kmax = -(-n_chunks // _NW)

    def body(x_hbm, tab_hbm, map_hbm, out_hbm, idx_v, x_v, g_v, sem):
        c = lax.axis_index("c")
        s = lax.axis_index("s")
        w = s * _NC + c

        def chunk(k, carry):
            cid = k * _NW + w

            @pl.when(cid < n_chunks)
            def _():
                pltpu.sync_copy(map_hbm.at[cid], idx_v)
                roff = pl.multiple_of(cid * CH, CH)
                d1 = pltpu.async_copy(tab_hbm.at[idx_v], g_v, sem)
                pltpu.sync_copy(x_hbm.at[pl.ds(roff, CH)], x_v)
                d1.wait()

                def row(r, carry2):
                    for j in range(8):
                        sl = pl.ds(j * 16, 16)
                        x_v[r, sl] = x_v[r, sl] + g_v[r, sl]
                    return carry2

                lax.fori_loop(0, CH, row, 0)
                pltpu.sync_copy(x_v, out_hbm.at[pl.ds(roff, CH)])

            return carry

        lax.fori_loop(0, kmax, chunk, 0)

    return pl.kernel(
        body,
        out_type=jax.ShapeDtypeStruct((n_rows, N_FILTERS), jnp.float32),
        mesh=_sc_mesh(),
        scratch_types=[
            pltpu.VMEM((CH,), jnp.int32),
            pltpu.VMEM((CH, N_FILTERS), jnp.float32),
            pltpu.VMEM((CH, N_FILTERS), jnp.float32),
            pltpu.SemaphoreType.DMA,
        ],
    )(x, table, map_ch)


def kernel(z, xyz, cg_xyz, mapping, nbr_list, CG_nbr_list, embed,
           W_ef1, b_ef1, W_ef2, b_ef2, W_nf, b_nf, W_u1, b_u1, W_u2, b_u2):
    z2d = z.astype(jnp.int32)[:, None]
    m2d = mapping.astype(jnp.int32)[:, None]

    src_a = nbr_list[:, 0].astype(jnp.int32).reshape(N_EDGES // _CH, _CH)
    dst_a = nbr_list[:, 1].astype(jnp.int32).reshape(N_EDGES // _CH, _CH)
    src_c = CG_nbr_list[:, 0].astype(jnp.int32).reshape(N_CG_EDGES // _CH, _CH)
    dst_c = CG_nbr_list[:, 1].astype(jnp.int32).reshape(N_CG_EDGES // _CH, _CH)
    xyz16 = jnp.zeros((N_ATOMS, 16), jnp.float32).at[:, :3].set(xyz.astype(jnp.float32))
    cg16 = jnp.zeros((N_CG, 16), jnp.float32).at[:, :3].set(cg_xyz.astype(jnp.float32))
    diff2_a = _edge_diff_sc(xyz16, src_a, dst_a)
    diff2_c = _edge_diff_sc(cg16, src_c, dst_c)
    map_ch = mapping.astype(jnp.int32).reshape(N_ATOMS // 80, 80)
    zeros_br = jnp.zeros((40, N_FILTERS), jnp.float32)
    zeros_a = jnp.zeros((1000, N_FILTERS), jnp.float32)
    zeros_c = jnp.zeros((200, N_FILTERS), jnp.float32)

    s_i = _embed(z2d, embed)
    counts = _counts(m2d)
    c_col = counts.T

    S_I = None
    f = _edge_filter(diff2_a, W_ef1[0], b_ef1[0][None, :], W_ef2[0], b_ef2[0][None, :])
    for i in range(N_CONV):
        j = N_CONV + i
        rn = _rn(s_i, W_nf[i], b_nf[i][None, :])
        agg2 = _msg_scatter_sc(rn, f, src_a, dst_a, zeros_a, N_ATOMS, N_EDGES // _CH)
        fc = _edge_filter(diff2_c, W_ef1[j], b_ef1[j][None, :], W_ef2[j], b_ef2[j][None, :])
        if i < N_CONV - 1:
            f = _edge_filter(diff2_a, W_ef1[i + 1], b_ef1[i + 1][None, :], W_ef2[i + 1], b_ef2[i + 1][None, :])
        s_i = _update(s_i, agg2[0], agg2[1], W_u1[i], b_u1[i][None, :], W_u2[i], b_u2[i][None, :])

        S_parts = _seg_sum_sc(s_i, map_ch, zeros_br, N_ATOMS, N_CG)
        S_input = _div(S_parts[0], S_parts[1], c_col)
        if i == 0:
            S_I = S_input

        Rn = _rn(S_input, W_nf[j], b_nf[j][None, :])
        Agg2 = _msg_scatter_sc(Rn, fc, src_c, dst_c, zeros_c, N_CG, N_CG_EDGES // _CH)
        S_I = _update(S_I, Agg2[0], Agg2[1], W_u1[j], b_u1[j][None, :], W_u2[j], b_u2[j][None, :])

        if i < N_CONV - 1:
            s_i = _gather_add_sc(s_i, S_I, map_ch, N_ATOMS)

    return S_I


# speedup vs baseline: 2.0839x; 1.1161x over previous
import functools
import numpy as np
import jax
import jax.numpy as jnp
from jax import lax
from jax.experimental import pallas as pl
from jax.experimental.pallas import tpu as pltpu
from jax.experimental.pallas import tpu_sc as plsc

N_ATOMS = 10000
N_CG = 1000
N_EDGES = 320000
N_CG_EDGES = 32000
N_ATOM_BASIS = 128
N_FILTERS = 128
N_GAUSSIANS = 50
N_CONV = 3
CUTOFF = 5.0

_OFFSETS = np.linspace(0.0, CUTOFF, N_GAUSSIANS).astype(np.float32)
_WIDTH = float(_OFFSETS[1] - _OFFSETS[0])
_COEFF = -0.5 / _WIDTH**2
_LOG2 = float(np.log(2.0))


def _ssp(x):
    return jnp.logaddexp(x, 0.0) - _LOG2


def _embed_body(z_ref, emb_ref, out_ref):
    z = z_ref[...]
    oh = (z == jax.lax.broadcasted_iota(jnp.int32, (1, 100), 1)).astype(jnp.float32)
    out_ref[...] = jnp.dot(oh, emb_ref[...], preferred_element_type=jnp.float32)


def _embed(z2d, embed):
    n = z2d.shape[0]
    blk = 1000
    return pl.pallas_call(
        _embed_body,
        grid=(n // blk,),
        in_specs=[
            pl.BlockSpec((blk, 1), lambda i: (i, 0)),
            pl.BlockSpec((100, N_ATOM_BASIS), lambda i: (0, 0)),
        ],
        out_specs=pl.BlockSpec((blk, N_ATOM_BASIS), lambda i: (i, 0)),
        out_shape=jax.ShapeDtypeStruct((n, N_ATOM_BASIS), jnp.float32),
    )(z2d, embed)


def _counts_body(m_ref, out_ref):
    i = pl.program_id(0)
    m = m_ref[...]
    oh = (m == jax.lax.broadcasted_iota(jnp.int32, (1, N_CG), 1)).astype(jnp.float32)
    c = jnp.sum(oh, axis=0, keepdims=True)

    @pl.when(i == 0)
    def _():
        out_ref[...] = jnp.zeros_like(out_ref)

    out_ref[...] += c


def _counts(m2d):
    n = m2d.shape[0]
    blk = 1000
    return pl.pallas_call(
        _counts_body,
        grid=(n // blk,),
        in_specs=[pl.BlockSpec((blk, 1), lambda i: (i, 0))],
        out_specs=pl.BlockSpec((1, N_CG), lambda i: (0, 0)),
        out_shape=jax.ShapeDtypeStruct((1, N_CG), jnp.float32),
    )(m2d)


def _edge_filter_body(g_ref, W1_ref, b1_ref, W2_ref, b2_ref, out_ref):
    d = jnp.sqrt(jnp.sum(g_ref[...], axis=1, keepdims=True))
    offs = jax.lax.broadcasted_iota(jnp.int32, (1, N_GAUSSIANS), 1).astype(jnp.float32) * (CUTOFF / (N_GAUSSIANS - 1))
    g = jnp.exp(_COEFF * (d - offs) ** 2)
    h = _ssp(jnp.dot(g, W1_ref[...], preferred_element_type=jnp.float32) + b1_ref[...])
    out_ref[...] = jnp.dot(h, W2_ref[...], preferred_element_type=jnp.float32) + b2_ref[...]


def _edge_filter(diff2, W1, b1, W2, b2):
    e = diff2.shape[0]
    blk = 2000
    return pl.pallas_call(
        _edge_filter_body,
        grid=(e // blk,),
        in_specs=[
            pl.BlockSpec((blk, 16), lambda i: (i, 0)),
            pl.BlockSpec((N_GAUSSIANS, N_GAUSSIANS), lambda i: (0, 0)),
            pl.BlockSpec((1, N_GAUSSIANS), lambda i: (0, 0)),
            pl.BlockSpec((N_GAUSSIANS, N_FILTERS), lambda i: (0, 0)),
            pl.BlockSpec((1, N_FILTERS), lambda i: (0, 0)),
        ],
        out_specs=pl.BlockSpec((blk, N_FILTERS), lambda i: (i, 0)),
        out_shape=jax.ShapeDtypeStruct((e, N_FILTERS), jnp.float32),
    )(diff2, W1, b1, W2, b2)


def _rn_body(s_ref, W_ref, b_ref, out_ref):
    out_ref[...] = jnp.dot(s_ref[...], W_ref[...], preferred_element_type=jnp.float32) + b_ref[...]


def _rn(s, W, b):
    n = s.shape[0]
    blk = min(n, 2000)
    return pl.pallas_call(
        _rn_body,
        grid=(n // blk,),
        in_specs=[
            pl.BlockSpec((blk, N_ATOM_BASIS), lambda i: (i, 0)),
            pl.BlockSpec((N_ATOM_BASIS, N_FILTERS), lambda i: (0, 0)),
            pl.BlockSpec((1, N_FILTERS), lambda i: (0, 0)),
        ],
        out_specs=pl.BlockSpec((blk, N_FILTERS), lambda i: (i, 0)),
        out_shape=jax.ShapeDtypeStruct((n, N_FILTERS), jnp.float32),
    )(s, W, b)


def _update_body(s_ref, a0_ref, a1_ref, W1_ref, b1_ref, W2_ref, b2_ref, out_ref):
    agg = a0_ref[...] + a1_ref[...]
    h = _ssp(jnp.dot(agg, W1_ref[...], preferred_element_type=jnp.float32) + b1_ref[...])
    out_ref[...] = s_ref[...] + jnp.dot(h, W2_ref[...], preferred_element_type=jnp.float32) + b2_ref[...]


def _update(s, a0, a1, Wu1, bu1, Wu2, bu2):
    n = s.shape[0]
    blk = min(n, 2000)
    return pl.pallas_call(
        _update_body,
        grid=(n // blk,),
        in_specs=[
            pl.BlockSpec((blk, N_ATOM_BASIS), lambda i: (i, 0)),
            pl.BlockSpec((blk, N_FILTERS), lambda i: (i, 0)),
            pl.BlockSpec((blk, N_FILTERS), lambda i: (i, 0)),
            pl.BlockSpec((N_FILTERS, N_ATOM_BASIS), lambda i: (0, 0)),
            pl.BlockSpec((1, N_ATOM_BASIS), lambda i: (0, 0)),
            pl.BlockSpec((N_ATOM_BASIS, N_ATOM_BASIS), lambda i: (0, 0)),
            pl.BlockSpec((1, N_ATOM_BASIS), lambda i: (0, 0)),
        ],
        out_specs=pl.BlockSpec((blk, N_ATOM_BASIS), lambda i: (i, 0)),
        out_shape=jax.ShapeDtypeStruct((n, N_ATOM_BASIS), jnp.float32),
    )(s, a0, a1, Wu1, bu1, Wu2, bu2)


def _div_body(S0_ref, S1_ref, c_ref, out_ref):
    c = jnp.maximum(c_ref[...], 1.0)
    out_ref[...] = (S0_ref[...] + S1_ref[...]) / c


def _div(S0, S1, c_col):
    return pl.pallas_call(
        _div_body,
        grid=(1,),
        in_specs=[
            pl.BlockSpec((N_CG, N_ATOM_BASIS), lambda i: (0, 0)),
            pl.BlockSpec((N_CG, N_ATOM_BASIS), lambda i: (0, 0)),
            pl.BlockSpec((N_CG, 1), lambda i: (0, 0)),
        ],
        out_specs=pl.BlockSpec((N_CG, N_ATOM_BASIS), lambda i: (0, 0)),
        out_shape=jax.ShapeDtypeStruct((N_CG, N_ATOM_BASIS), jnp.float32),
    )(S0, S1, c_col)


_NC, _NS = 2, 16
_NW = _NC * _NS


def _sc_mesh():
    return plsc.VectorSubcoreMesh(
        core_axis_name="c", subcore_axis_name="s", num_cores=_NC, num_subcores=_NS)


_CH = 40
_G = 32
_DCH = 80


def _msg_scatter_sc(rn, f_ch, src_ch, dst_ch, zeros, n_nodes, n_chunks):
    N = n_nodes
    BR = zeros.shape[0]
    nbl = N // BR
    kz = -(-nbl // _NS)
    cpw = -(-n_chunks // _NW)
    cpw = -(-cpw // _G) * _G
    ngr = cpw // _G

    cpad = _NW * cpw
    src_p = jnp.zeros((cpad, _CH), jnp.int32).at[:n_chunks].set(src_ch)
    dst_p = jnp.zeros((cpad, _CH), jnp.int32).at[:n_chunks].set(dst_ch)

    def body(rn_hbm, f_hbm, src_hbm, dst_hbm, z_hbm, out_hbm,
             agg_sh, isa_v, ida_v, isb_v, idb_v,
             fa_v, rsa_v, rda_v, fb_v, rsb_v, rdb_v, sema, semb, semi):
        c = lax.axis_index("c")
        s = lax.axis_index("s")
        w = s * _NC + c
        base = w * cpw
        nloc = jnp.minimum(jnp.maximum(n_chunks - base, 0), cpw)

        def zero_blk(k2, carry):
            bid = k2 * _NS + s

            @pl.when(bid < nbl)
            def _():
                off = pl.multiple_of(bid * BR, BR)
                pltpu.sync_copy(z_hbm, agg_sh.at[pl.ds(off, BR)])

            return carry

        def fetch_idx(g, is_v, id_v):
            goff = pl.multiple_of(base + g * _G, _G)
            pltpu.async_copy(src_hbm.at[pl.ds(goff, _G)], is_v, semi)
            pltpu.async_copy(dst_hbm.at[pl.ds(goff, _G)], id_v, semi)

        def drain_idx(is_v, id_v):
            pltpu.make_async_copy(src_hbm.at[pl.ds(0, _G)], is_v, semi).wait()
            pltpu.make_async_copy(src_hbm.at[pl.ds(0, _G)], id_v, semi).wait()

        fetch_idx(0, isa_v, ida_v)
        lax.fori_loop(0, kz, zero_blk, 0)
        plsc.subcore_barrier()

        def fire(g, t, is_v, id_v, f_v, rs_v, rd_v, sem):
            j = g * _G + t

            @pl.when(j < nloc)
            def _():
                foff = pl.multiple_of((base + j) * _CH, _CH)
                pltpu.async_copy(f_hbm.at[pl.ds(foff, _CH)], f_v, sem)
                pltpu.async_copy(rn_hbm.at[is_v.at[t]], rs_v, sem)
                pltpu.async_copy(rn_hbm.at[id_v.at[t]], rd_v, sem)

        def drain_compute(g, t, is_v, id_v, f_v, rs_v, rd_v, sem):
            j = g * _G + t

            @pl.when(j < nloc)
            def _():
                pltpu.make_async_copy(f_hbm.at[pl.ds(0, _CH)], f_v, sem).wait()
                pltpu.make_async_copy(f_hbm.at[pl.ds(0, _CH)], rs_v, sem).wait()
                pltpu.make_async_copy(f_hbm.at[pl.ds(0, _CH)], rd_v, sem).wait()

                @plsc.parallel_loop(0, _CH, unroll=8)
                def _(r):
                    for jj in range(8):
                        sl = pl.ds(jj * 16, 16)
                        fv = f_v[r, sl]
                        rs_v[r, sl] = rs_v[r, sl] * fv
                        rd_v[r, sl] = rd_v[r, sl] * fv

                pltpu.sync_copy(rs_v, agg_sh.at[id_v.at[t]], add=True)
                pltpu.sync_copy(rd_v, agg_sh.at[is_v.at[t]], add=True)

        def group(g, carry):
            geven = g % 2 == 0

            def run(is_v, id_v, isn_v, idn_v):
                drain_idx(is_v, id_v)

                @pl.when(g + 1 < ngr)
                def _():
                    fetch_idx(g + 1, isn_v, idn_v)

                fire(g, 0, is_v, id_v, fa_v, rsa_v, rda_v, sema)

                def pair(t2, carry2):
                    t0 = t2 * 2
                    fire(g, t0 + 1, is_v, id_v, fb_v, rsb_v, rdb_v, semb)
                    drain_compute(g, t0, is_v, id_v, fa_v, rsa_v, rda_v, sema)

                    @pl.when(t0 + 2 < _G)
                    def _():
                        fire(g, t0 + 2, is_v, id_v, fa_v, rsa_v, rda_v, sema)

                    drain_compute(g, t0 + 1, is_v, id_v, fb_v, rsb_v, rdb_v,
                                  semb)
                    return carry2

                lax.fori_loop(0, _G // 2, pair, 0)

            @pl.when(geven)
            def _():
                run(isa_v, ida_v, isb_v, idb_v)

            @pl.when(jnp.logical_not(geven))
            def _():
                run(isb_v, idb_v, isa_v, ida_v)

            return carry

        lax.fori_loop(0, ngr, group, 0)
        plsc.subcore_barrier()

        def read_blk(k2, carry):
            bid = k2 * _NS + s

            @pl.when(bid < nbl)
            def _():
                off = pl.multiple_of(bid * BR, BR)
                pltpu.sync_copy(agg_sh.at[pl.ds(off, BR)],
                                out_hbm.at[c, pl.ds(off, BR)])

            return carry

        lax.fori_loop(0, kz, read_blk, 0)

    return pl.kernel(
        body,
        out_type=jax.ShapeDtypeStruct((_NC, N, N_FILTERS), jnp.float32),
        mesh=_sc_mesh(),
        scratch_types=[
            pltpu.VMEM_SHARED((N, N_FILTERS), jnp.float32),
            pltpu.VMEM((_G, _CH), jnp.int32),
            pltpu.VMEM((_G, _CH), jnp.int32),
            pltpu.VMEM((_G, _CH), jnp.int32),
            pltpu.VMEM((_G, _CH), jnp.int32),
            pltpu.VMEM((_CH, N_FILTERS), jnp.float32),
            pltpu.VMEM((_CH, N_FILTERS), jnp.float32),
            pltpu.VMEM((_CH, N_FILTERS), jnp.float32),
            pltpu.VMEM((_CH, N_FILTERS), jnp.float32),
            pltpu.VMEM((_CH, N_FILTERS), jnp.float32),
            pltpu.VMEM((_CH, N_FILTERS), jnp.float32),
            pltpu.SemaphoreType.DMA,
            pltpu.SemaphoreType.DMA,
            pltpu.SemaphoreType.DMA,
        ],
    )(rn, f_ch, src_p, dst_p, zeros)


def _seg_sum_sc(x, map_ch, zeros, n_rows, n_out):
    CH = 80
    n_chunks = n_rows // CH
    kmax = -(-n_chunks // _NW)
    BR = 40
    nbl = n_out // BR
    kz = -(-nbl // _NS)

    def body(x_hbm, map_hbm, z_hbm, out_hbm, acc_sh, idx_v, x_v, sem):
        c = lax.axis_index("c")
        s = lax.axis_index("s")
        w = s * _NC + c

        def zero_blk(k2, carry):
            bid = k2 * _NS + s

            @pl.when(bid < nbl)
            def _():
                off = pl.multiple_of(bid * BR, BR)
                pltpu.sync_copy(z_hbm, acc_sh.at[pl.ds(off, BR)])

            return carry

        lax.fori_loop(0, kz, zero_blk, 0)
        plsc.subcore_barrier()

        def chunk(k, carry):
            cid = k * _NW + w

            @pl.when(cid < n_chunks)
            def _():
                pltpu.sync_copy(map_hbm.at[cid], idx_v)
                roff = pl.multiple_of(cid * CH, CH)
                pltpu.sync_copy(x_hbm.at[pl.ds(roff, CH)], x_v)
                pltpu.sync_copy(x_v, acc_sh.at[idx_v], add=True)

            return carry

        lax.fori_loop(0, kmax, chunk, 0)
        plsc.subcore_barrier()

        def read_blk(k2, carry):
            bid = k2 * _NS + s

            @pl.when(bid < nbl)
            def _():
                off = pl.multiple_of(bid * BR, BR)
                pltpu.sync_copy(acc_sh.at[pl.ds(off, BR)],
                                out_hbm.at[c, pl.ds(off, BR)])

            return carry

        lax.fori_loop(0, kz, read_blk, 0)

    return pl.kernel(
        body,
        out_type=jax.ShapeDtypeStruct((_NC, n_out, N_FILTERS), jnp.float32),
        mesh=_sc_mesh(),
        scratch_types=[
            pltpu.VMEM_SHARED((n_out, N_FILTERS), jnp.float32),
            pltpu.VMEM((CH,), jnp.int32),
            pltpu.VMEM((CH, N_FILTERS), jnp.float32),
            pltpu.SemaphoreType.DMA,
        ],
    )(x, map_ch, zeros)


def _edge_diff_sc(xyz16, src_ch, dst_ch):
    n_chunks = src_ch.shape[0]
    kmax = -(-n_chunks // _NW)

    def body(xyz_hbm, src_hbm, dst_hbm, out_hbm,
             isa, ida, isb, idb, aa, ba, ab, bb, sema, semb):
        c = lax.axis_index("c")
        s = lax.axis_index("s")
        w = s * _NC + c

        def fire(k, is_v, id_v, a_v, b_v, sem):
            cid = k * _NW + w

            @pl.when(cid < n_chunks)
            def _():
                pltpu.sync_copy(src_hbm.at[cid], is_v)
                pltpu.sync_copy(dst_hbm.at[cid], id_v)
                pltpu.async_copy(xyz_hbm.at[is_v], a_v, sem)
                pltpu.async_copy(xyz_hbm.at[id_v], b_v, sem)

        def drain(k, is_v, id_v, a_v, b_v, sem):
            cid = k * _NW + w

            @pl.when(cid < n_chunks)
            def _():
                pltpu.make_async_copy(xyz_hbm.at[is_v], a_v, sem).wait()
                pltpu.make_async_copy(xyz_hbm.at[id_v], b_v, sem).wait()

                @plsc.parallel_loop(0, _DCH, unroll=8)
                def _(r):
                    d = a_v[r, :] - b_v[r, :]
                    a_v[r, :] = d * d

                roff = pl.multiple_of(cid * _DCH, _DCH)
                pltpu.sync_copy(a_v, out_hbm.at[pl.ds(roff, _DCH)])

        fire(0, isa, ida, aa, ba, sema)

        def chunk(k, carry):
            keven = k % 2 == 0

            @pl.when(keven)
            def _():
                fire(k + 1, isb, idb, ab, bb, semb)
                drain(k, isa, ida, aa, ba, sema)

            @pl.when(jnp.logical_not(keven))
            def _():
                fire(k + 1, isa, ida, aa, ba, sema)
                drain(k, isb, idb, ab, bb, semb)

            return carry

        lax.fori_loop(0, kmax, chunk, 0)

    return pl.kernel(
        body,
        out_type=jax.ShapeDtypeStruct((n_chunks * _DCH, 16), jnp.float32),
        mesh=_sc_mesh(),
        compiler_params=pltpu.CompilerParams(use_tc_tiling_on_sc=False),
        scratch_types=[
            pltpu.VMEM((_DCH,), jnp.int32),
            pltpu.VMEM((_DCH,), jnp.int32),
            pltpu.VMEM((_DCH,), jnp.int32),
            pltpu.VMEM((_DCH,), jnp.int32),
            pltpu.VMEM((_DCH, 16), jnp.float32),
            pltpu.VMEM((_DCH, 16), jnp.float32),
            pltpu.VMEM((_DCH, 16), jnp.float32),
            pltpu.VMEM((_DCH, 16), jnp.float32),
            pltpu.SemaphoreType.DMA,
            pltpu.SemaphoreType.DMA,
        ],
    )(xyz16, src_ch, dst_ch)


def _gather_add_sc(x, table, map_ch, n_rows):
    CH = 80
    n_chunks = n_rows // CH
    kmax = -(-n_chunks // _NW)

    def body(x_hbm, tab_hbm, map_hbm, out_hbm, idx_v, x_v, g_v, sem):
        c = lax.axis_index("c")
        s = lax.axis_index("s")
        w = s * _NC + c

        def chunk(k, carry):
            cid = k * _NW + w

            @pl.when(cid < n_chunks)
            def _():
                pltpu.sync_copy(map_hbm.at[cid], idx_v)
                roff = pl.multiple_of(cid * CH, CH)
                d1 = pltpu.async_copy(tab_hbm.at[idx_v], g_v, sem)
                pltpu.sync_copy(x_hbm.at[pl.ds(roff, CH)], x_v)
                d1.wait()

                def row(r, carry2):
                    for j in range(8):
                        sl = pl.ds(j * 16, 16)
                        x_v[r, sl] = x_v[r, sl] + g_v[r, sl]
                    return carry2

                lax.fori_loop(0, CH, row, 0)
                pltpu.sync_copy(x_v, out_hbm.at[pl.ds(roff, CH)])

            return carry

        lax.fori_loop(0, kmax, chunk, 0)

    return pl.kernel(
        body,
        out_type=jax.ShapeDtypeStruct((n_rows, N_FILTERS), jnp.float32),
        mesh=_sc_mesh(),
        scratch_types=[
            pltpu.VMEM((CH,), jnp.int32),
            pltpu.VMEM((CH, N_FILTERS), jnp.float32),
            pltpu.VMEM((CH, N_FILTERS), jnp.float32),
            pltpu.SemaphoreType.DMA,
        ],
    )(x, table, map_ch)


def kernel(z, xyz, cg_xyz, mapping, nbr_list, CG_nbr_list, embed,
           W_ef1, b_ef1, W_ef2, b_ef2, W_nf, b_nf, W_u1, b_u1, W_u2, b_u2):
    z2d = z.astype(jnp.int32)[:, None]
    m2d = mapping.astype(jnp.int32)[:, None]

    src_a = nbr_list[:, 0].astype(jnp.int32).reshape(N_EDGES // _CH, _CH)
    dst_a = nbr_list[:, 1].astype(jnp.int32).reshape(N_EDGES // _CH, _CH)
    src_c = CG_nbr_list[:, 0].astype(jnp.int32).reshape(N_CG_EDGES // _CH, _CH)
    dst_c = CG_nbr_list[:, 1].astype(jnp.int32).reshape(N_CG_EDGES // _CH, _CH)
    xyz16 = jnp.zeros((N_ATOMS, 16), jnp.float32).at[:, :3].set(xyz.astype(jnp.float32))
    cg16 = jnp.zeros((N_CG, 16), jnp.float32).at[:, :3].set(cg_xyz.astype(jnp.float32))
    diff2_a = _edge_diff_sc(
        xyz16,
        src_a.reshape(N_EDGES // _DCH, _DCH),
        dst_a.reshape(N_EDGES // _DCH, _DCH))
    diff2_c = _edge_diff_sc(
        cg16,
        src_c.reshape(N_CG_EDGES // _DCH, _DCH),
        dst_c.reshape(N_CG_EDGES // _DCH, _DCH))
    map_ch = mapping.astype(jnp.int32).reshape(N_ATOMS // 80, 80)
    zeros_br = jnp.zeros((40, N_FILTERS), jnp.float32)
    zeros_a = jnp.zeros((1000, N_FILTERS), jnp.float32)
    zeros_c = jnp.zeros((200, N_FILTERS), jnp.float32)

    s_i = _embed(z2d, embed)
    counts = _counts(m2d)
    c_col = counts.T

    S_I = None
    f = _edge_filter(diff2_a, W_ef1[0], b_ef1[0][None, :], W_ef2[0], b_ef2[0][None, :])
    for i in range(N_CONV):
        j = N_CONV + i
        rn = _rn(s_i, W_nf[i], b_nf[i][None, :])
        agg2 = _msg_scatter_sc(rn, f, src_a, dst_a, zeros_a, N_ATOMS, N_EDGES // _CH)
        fc = _edge_filter(diff2_c, W_ef1[j], b_ef1[j][None, :], W_ef2[j], b_ef2[j][None, :])
        if i < N_CONV - 1:
            f = _edge_filter(diff2_a, W_ef1[i + 1], b_ef1[i + 1][None, :], W_ef2[i + 1], b_ef2[i + 1][None, :])
        s_i = _update(s_i, agg2[0], agg2[1], W_u1[i], b_u1[i][None, :], W_u2[i], b_u2[i][None, :])

        S_parts = _seg_sum_sc(s_i, map_ch, zeros_br, N_ATOMS, N_CG)
        S_input = _div(S_parts[0], S_parts[1], c_col)
        if i == 0:
            S_I = S_input

        Rn = _rn(S_input, W_nf[j], b_nf[j][None, :])
        Agg2 = _msg_scatter_sc(Rn, fc, src_c, dst_c, zeros_c, N_CG, N_CG_EDGES // _CH)
        S_I = _update(S_I, Agg2[0], Agg2[1], W_u1[j], b_u1[j][None, :], W_u2[j], b_u2[j][None, :])

        if i < N_CONV - 1:
            s_i = _gather_add_sc(s_i, S_I, map_ch, N_ATOMS)

    return S_I


# speedup vs baseline: 2.0856x; 1.0008x over previous
import functools
import numpy as np
import jax
import jax.numpy as jnp
from jax import lax
from jax.experimental import pallas as pl
from jax.experimental.pallas import tpu as pltpu
from jax.experimental.pallas import tpu_sc as plsc

N_ATOMS = 10000
N_CG = 1000
N_EDGES = 320000
N_CG_EDGES = 32000
N_ATOM_BASIS = 128
N_FILTERS = 128
N_GAUSSIANS = 50
N_CONV = 3
CUTOFF = 5.0

_OFFSETS = np.linspace(0.0, CUTOFF, N_GAUSSIANS).astype(np.float32)
_WIDTH = float(_OFFSETS[1] - _OFFSETS[0])
_COEFF = -0.5 / _WIDTH**2
_LOG2 = float(np.log(2.0))


def _ssp(x):
    return jnp.logaddexp(x, 0.0) - _LOG2


def _embed_body(z_ref, emb_ref, out_ref):
    z = z_ref[...]
    oh = (z == jax.lax.broadcasted_iota(jnp.int32, (1, 100), 1)).astype(jnp.float32)
    out_ref[...] = jnp.dot(oh, emb_ref[...], preferred_element_type=jnp.float32)


def _embed(z2d, embed):
    n = z2d.shape[0]
    blk = 1000
    return pl.pallas_call(
        _embed_body,
        grid=(n // blk,),
        in_specs=[
            pl.BlockSpec((blk, 1), lambda i: (i, 0)),
            pl.BlockSpec((100, N_ATOM_BASIS), lambda i: (0, 0)),
        ],
        out_specs=pl.BlockSpec((blk, N_ATOM_BASIS), lambda i: (i, 0)),
        out_shape=jax.ShapeDtypeStruct((n, N_ATOM_BASIS), jnp.float32),
    )(z2d, embed)


def _counts_body(m_ref, out_ref):
    i = pl.program_id(0)
    m = m_ref[...]
    oh = (m == jax.lax.broadcasted_iota(jnp.int32, (1, N_CG), 1)).astype(jnp.float32)
    c = jnp.sum(oh, axis=0, keepdims=True)

    @pl.when(i == 0)
    def _():
        out_ref[...] = jnp.zeros_like(out_ref)

    out_ref[...] += c


def _counts(m2d):
    n = m2d.shape[0]
    blk = 1000
    return pl.pallas_call(
        _counts_body,
        grid=(n // blk,),
        in_specs=[pl.BlockSpec((blk, 1), lambda i: (i, 0))],
        out_specs=pl.BlockSpec((1, N_CG), lambda i: (0, 0)),
        out_shape=jax.ShapeDtypeStruct((1, N_CG), jnp.float32),
    )(m2d)


def _edge_filter_body(g_ref, W1_ref, b1_ref, W2_ref, b2_ref, out_ref):
    d = jnp.sqrt(jnp.sum(g_ref[...], axis=1, keepdims=True))
    offs = jax.lax.broadcasted_iota(jnp.int32, (1, N_GAUSSIANS), 1).astype(jnp.float32) * (CUTOFF / (N_GAUSSIANS - 1))
    g = jnp.exp(_COEFF * (d - offs) ** 2)
    h = _ssp(jnp.dot(g, W1_ref[...], preferred_element_type=jnp.float32) + b1_ref[...])
    out_ref[...] = jnp.dot(h, W2_ref[...], preferred_element_type=jnp.float32) + b2_ref[...]


def _edge_filter(diff2, W1, b1, W2, b2):
    e = diff2.shape[0]
    blk = 2000
    return pl.pallas_call(
        _edge_filter_body,
        grid=(e // blk,),
        in_specs=[
            pl.BlockSpec((blk, 16), lambda i: (i, 0)),
            pl.BlockSpec((N_GAUSSIANS, N_GAUSSIANS), lambda i: (0, 0)),
            pl.BlockSpec((1, N_GAUSSIANS), lambda i: (0, 0)),
            pl.BlockSpec((N_GAUSSIANS, N_FILTERS), lambda i: (0, 0)),
            pl.BlockSpec((1, N_FILTERS), lambda i: (0, 0)),
        ],
        out_specs=pl.BlockSpec((blk, N_FILTERS), lambda i: (i, 0)),
        out_shape=jax.ShapeDtypeStruct((e, N_FILTERS), jnp.float32),
    )(diff2, W1, b1, W2, b2)


def _rn_body(s_ref, W_ref, b_ref, out_ref):
    out_ref[...] = jnp.dot(s_ref[...], W_ref[...], preferred_element_type=jnp.float32) + b_ref[...]


def _rn(s, W, b):
    n = s.shape[0]
    blk = min(n, 2000)
    return pl.pallas_call(
        _rn_body,
        grid=(n // blk,),
        in_specs=[
            pl.BlockSpec((blk, N_ATOM_BASIS), lambda i: (i, 0)),
            pl.BlockSpec((N_ATOM_BASIS, N_FILTERS), lambda i: (0, 0)),
            pl.BlockSpec((1, N_FILTERS), lambda i: (0, 0)),
        ],
        out_specs=pl.BlockSpec((blk, N_FILTERS), lambda i: (i, 0)),
        out_shape=jax.ShapeDtypeStruct((n, N_FILTERS), jnp.float32),
    )(s, W, b)


def _update_body(s_ref, a0_ref, a1_ref, W1_ref, b1_ref, W2_ref, b2_ref, out_ref):
    agg = a0_ref[...] + a1_ref[...]
    h = _ssp(jnp.dot(agg, W1_ref[...], preferred_element_type=jnp.float32) + b1_ref[...])
    out_ref[...] = s_ref[...] + jnp.dot(h, W2_ref[...], preferred_element_type=jnp.float32) + b2_ref[...]


def _update(s, a0, a1, Wu1, bu1, Wu2, bu2):
    n = s.shape[0]
    blk = min(n, 2000)
    return pl.pallas_call(
        _update_body,
        grid=(n // blk,),
        in_specs=[
            pl.BlockSpec((blk, N_ATOM_BASIS), lambda i: (i, 0)),
            pl.BlockSpec((blk, N_FILTERS), lambda i: (i, 0)),
            pl.BlockSpec((blk, N_FILTERS), lambda i: (i, 0)),
            pl.BlockSpec((N_FILTERS, N_ATOM_BASIS), lambda i: (0, 0)),
            pl.BlockSpec((1, N_ATOM_BASIS), lambda i: (0, 0)),
            pl.BlockSpec((N_ATOM_BASIS, N_ATOM_BASIS), lambda i: (0, 0)),
            pl.BlockSpec((1, N_ATOM_BASIS), lambda i: (0, 0)),
        ],
        out_specs=pl.BlockSpec((blk, N_ATOM_BASIS), lambda i: (i, 0)),
        out_shape=jax.ShapeDtypeStruct((n, N_ATOM_BASIS), jnp.float32),
    )(s, a0, a1, Wu1, bu1, Wu2, bu2)


def _div_body(S0_ref, S1_ref, c_ref, out_ref):
    c = jnp.maximum(c_ref[...], 1.0)
    out_ref[...] = (S0_ref[...] + S1_ref[...]) / c


def _div(S0, S1, c_col):
    return pl.pallas_call(
        _div_body,
        grid=(1,),
        in_specs=[
            pl.BlockSpec((N_CG, N_ATOM_BASIS), lambda i: (0, 0)),
            pl.BlockSpec((N_CG, N_ATOM_BASIS), lambda i: (0, 0)),
            pl.BlockSpec((N_CG, 1), lambda i: (0, 0)),
        ],
        out_specs=pl.BlockSpec((N_CG, N_ATOM_BASIS), lambda i: (0, 0)),
        out_shape=jax.ShapeDtypeStruct((N_CG, N_ATOM_BASIS), jnp.float32),
    )(S0, S1, c_col)


_NC, _NS = 2, 16
_NW = _NC * _NS


def _sc_mesh():
    return plsc.VectorSubcoreMesh(
        core_axis_name="c", subcore_axis_name="s", num_cores=_NC, num_subcores=_NS)


_CH = 40
_G = 32
_DCH = 80


def _msg_scatter_sc(rn, f_ch, src_ch, dst_ch, zeros, n_nodes, n_chunks):
    N = n_nodes
    BR = zeros.shape[0]
    nbl = N // BR
    kz = -(-nbl // _NS)
    cpw = -(-n_chunks // _NW)
    cpw = -(-cpw // _G) * _G
    ngr = cpw // _G

    cpad = _NW * cpw
    src_p = jnp.zeros((cpad, _CH), jnp.int32).at[:n_chunks].set(src_ch)
    dst_p = jnp.zeros((cpad, _CH), jnp.int32).at[:n_chunks].set(dst_ch)

    def body(rn_hbm, f_hbm, src_hbm, dst_hbm, z_hbm, out_hbm,
             agg_sh, isa_v, ida_v, isb_v, idb_v,
             fa_v, rsa_v, rda_v, fb_v, rsb_v, rdb_v, sema, semb, semi):
        c = lax.axis_index("c")
        s = lax.axis_index("s")
        w = s * _NC + c
        base = w * cpw
        nloc = jnp.minimum(jnp.maximum(n_chunks - base, 0), cpw)

        def zero_blk(k2, carry):
            bid = k2 * _NS + s

            @pl.when(bid < nbl)
            def _():
                off = pl.multiple_of(bid * BR, BR)
                pltpu.sync_copy(z_hbm, agg_sh.at[pl.ds(off, BR)])

            return carry

        def fetch_idx(g, is_v, id_v):
            goff = pl.multiple_of(base + g * _G, _G)
            pltpu.async_copy(src_hbm.at[pl.ds(goff, _G)], is_v, semi)
            pltpu.async_copy(dst_hbm.at[pl.ds(goff, _G)], id_v, semi)

        def drain_idx(is_v, id_v):
            pltpu.make_async_copy(src_hbm.at[pl.ds(0, _G)], is_v, semi).wait()
            pltpu.make_async_copy(src_hbm.at[pl.ds(0, _G)], id_v, semi).wait()

        fetch_idx(0, isa_v, ida_v)
        lax.fori_loop(0, kz, zero_blk, 0)
        plsc.subcore_barrier()

        def fire(g, t, is_v, id_v, f_v, rs_v, rd_v, sem):
            j = g * _G + t

            @pl.when(j < nloc)
            def _():
                foff = pl.multiple_of((base + j) * _CH, _CH)
                pltpu.async_copy(f_hbm.at[pl.ds(foff, _CH)], f_v, sem)
                pltpu.async_copy(rn_hbm.at[is_v.at[t]], rs_v, sem)
                pltpu.async_copy(rn_hbm.at[id_v.at[t]], rd_v, sem)

        def drain_compute(g, t, is_v, id_v, f_v, rs_v, rd_v, sem):
            j = g * _G + t

            @pl.when(j < nloc)
            def _():
                pltpu.make_async_copy(f_hbm.at[pl.ds(0, _CH)], f_v, sem).wait()
                pltpu.make_async_copy(f_hbm.at[pl.ds(0, _CH)], rs_v, sem).wait()
                pltpu.make_async_copy(f_hbm.at[pl.ds(0, _CH)], rd_v, sem).wait()

                @plsc.parallel_loop(0, _CH, unroll=8)
                def _(r):
                    for jj in range(8):
                        sl = pl.ds(jj * 16, 16)
                        fv = f_v[r, sl]
                        rs_v[r, sl] = rs_v[r, sl] * fv
                        rd_v[r, sl] = rd_v[r, sl] * fv

                pltpu.sync_copy(rs_v, agg_sh.at[id_v.at[t]], add=True)
                pltpu.sync_copy(rd_v, agg_sh.at[is_v.at[t]], add=True)

        def group(g, carry):
            geven = g % 2 == 0

            def run(is_v, id_v, isn_v, idn_v):
                drain_idx(is_v, id_v)

                @pl.when(g + 1 < ngr)
                def _():
                    fetch_idx(g + 1, isn_v, idn_v)

                fire(g, 0, is_v, id_v, fa_v, rsa_v, rda_v, sema)

                def pair(t2, carry2):
                    t0 = t2 * 2
                    fire(g, t0 + 1, is_v, id_v, fb_v, rsb_v, rdb_v, semb)
                    drain_compute(g, t0, is_v, id_v, fa_v, rsa_v, rda_v, sema)

                    @pl.when(t0 + 2 < _G)
                    def _():
                        fire(g, t0 + 2, is_v, id_v, fa_v, rsa_v, rda_v, sema)

                    drain_compute(g, t0 + 1, is_v, id_v, fb_v, rsb_v, rdb_v,
                                  semb)
                    return carry2

                lax.fori_loop(0, _G // 2, pair, 0)

            @pl.when(geven)
            def _():
                run(isa_v, ida_v, isb_v, idb_v)

            @pl.when(jnp.logical_not(geven))
            def _():
                run(isb_v, idb_v, isa_v, ida_v)

            return carry

        lax.fori_loop(0, ngr, group, 0)
        plsc.subcore_barrier()

        def read_blk(k2, carry):
            bid = k2 * _NS + s

            @pl.when(bid < nbl)
            def _():
                off = pl.multiple_of(bid * BR, BR)
                pltpu.sync_copy(agg_sh.at[pl.ds(off, BR)],
                                out_hbm.at[c, pl.ds(off, BR)])

            return carry

        lax.fori_loop(0, kz, read_blk, 0)

    return pl.kernel(
        body,
        out_type=jax.ShapeDtypeStruct((_NC, N, N_FILTERS), jnp.float32),
        mesh=_sc_mesh(),
        scratch_types=[
            pltpu.VMEM_SHARED((N, N_FILTERS), jnp.float32),
            pltpu.VMEM((_G, _CH), jnp.int32),
            pltpu.VMEM((_G, _CH), jnp.int32),
            pltpu.VMEM((_G, _CH), jnp.int32),
            pltpu.VMEM((_G, _CH), jnp.int32),
            pltpu.VMEM((_CH, N_FILTERS), jnp.float32),
            pltpu.VMEM((_CH, N_FILTERS), jnp.float32),
            pltpu.VMEM((_CH, N_FILTERS), jnp.float32),
            pltpu.VMEM((_CH, N_FILTERS), jnp.float32),
            pltpu.VMEM((_CH, N_FILTERS), jnp.float32),
            pltpu.VMEM((_CH, N_FILTERS), jnp.float32),
            pltpu.SemaphoreType.DMA,
            pltpu.SemaphoreType.DMA,
            pltpu.SemaphoreType.DMA,
        ],
    )(rn, f_ch, src_p, dst_p, zeros)


def _seg_sum_sc(x, map_ch, zeros, n_rows, n_out):
    CH = 80
    n_chunks = n_rows // CH
    kmax = -(-n_chunks // _NW)
    BR = 40
    nbl = n_out // BR
    kz = -(-nbl // _NS)

    def body(x_hbm, map_hbm, z_hbm, out_hbm, acc_sh, idx_v, x_v, sem):
        c = lax.axis_index("c")
        s = lax.axis_index("s")
        w = s * _NC + c

        def zero_blk(k2, carry):
            bid = k2 * _NS + s

            @pl.when(bid < nbl)
            def _():
                off = pl.multiple_of(bid * BR, BR)
                pltpu.sync_copy(z_hbm, acc_sh.at[pl.ds(off, BR)])

            return carry

        lax.fori_loop(0, kz, zero_blk, 0)
        plsc.subcore_barrier()

        def chunk(k, carry):
            cid = k * _NW + w

            @pl.when(cid < n_chunks)
            def _():
                pltpu.sync_copy(map_hbm.at[cid], idx_v)
                roff = pl.multiple_of(cid * CH, CH)
                pltpu.sync_copy(x_hbm.at[pl.ds(roff, CH)], x_v)
                pltpu.sync_copy(x_v, acc_sh.at[idx_v], add=True)

            return carry

        lax.fori_loop(0, kmax, chunk, 0)
        plsc.subcore_barrier()

        def read_blk(k2, carry):
            bid = k2 * _NS + s

            @pl.when(bid < nbl)
            def _():
                off = pl.multiple_of(bid * BR, BR)
                pltpu.sync_copy(acc_sh.at[pl.ds(off, BR)],
                                out_hbm.at[c, pl.ds(off, BR)])

            return carry

        lax.fori_loop(0, kz, read_blk, 0)

    return pl.kernel(
        body,
        out_type=jax.ShapeDtypeStruct((_NC, n_out, N_FILTERS), jnp.float32),
        mesh=_sc_mesh(),
        scratch_types=[
            pltpu.VMEM_SHARED((n_out, N_FILTERS), jnp.float32),
            pltpu.VMEM((CH,), jnp.int32),
            pltpu.VMEM((CH, N_FILTERS), jnp.float32),
            pltpu.SemaphoreType.DMA,
        ],
    )(x, map_ch, zeros)


def _edge_diff_sc(xyz16, src_ch, dst_ch):
    n_chunks = src_ch.shape[0]
    kmax = -(-n_chunks // _NW)

    def body(xyz_hbm, src_hbm, dst_hbm, out_hbm,
             isa, ida, isb, idb, aa, ba, ab, bb, sema, semb):
        c = lax.axis_index("c")
        s = lax.axis_index("s")
        w = s * _NC + c

        def fire(k, is_v, id_v, a_v, b_v, sem):
            cid = k * _NW + w

            @pl.when(cid < n_chunks)
            def _():
                pltpu.sync_copy(src_hbm.at[cid], is_v)
                pltpu.sync_copy(dst_hbm.at[cid], id_v)
                pltpu.async_copy(xyz_hbm.at[is_v], a_v, sem)
                pltpu.async_copy(xyz_hbm.at[id_v], b_v, sem)

        def drain(k, is_v, id_v, a_v, b_v, sem):
            cid = k * _NW + w

            @pl.when(cid < n_chunks)
            def _():
                pltpu.make_async_copy(xyz_hbm.at[pl.ds(0, _DCH)], a_v, sem).wait()
                pltpu.make_async_copy(xyz_hbm.at[pl.ds(0, _DCH)], b_v, sem).wait()

                @plsc.parallel_loop(0, _DCH, unroll=8)
                def _(r):
                    d = a_v[r, :] - b_v[r, :]
                    a_v[r, :] = d * d

                roff = pl.multiple_of(cid * _DCH, _DCH)
                pltpu.sync_copy(a_v, out_hbm.at[pl.ds(roff, _DCH)])

        fire(0, isa, ida, aa, ba, sema)

        def chunk(k, carry):
            keven = k % 2 == 0

            @pl.when(keven)
            def _():
                fire(k + 1, isb, idb, ab, bb, semb)
                drain(k, isa, ida, aa, ba, sema)

            @pl.when(jnp.logical_not(keven))
            def _():
                fire(k + 1, isa, ida, aa, ba, sema)
                drain(k, isb, idb, ab, bb, semb)

            return carry

        lax.fori_loop(0, kmax, chunk, 0)

    return pl.kernel(
        body,
        out_type=jax.ShapeDtypeStruct((n_chunks * _DCH, 16), jnp.float32),
        mesh=_sc_mesh(),
        compiler_params=pltpu.CompilerParams(use_tc_tiling_on_sc=False),
        scratch_types=[
            pltpu.VMEM((_DCH,), jnp.int32),
            pltpu.VMEM((_DCH,), jnp.int32),
            pltpu.VMEM((_DCH,), jnp.int32),
            pltpu.VMEM((_DCH,), jnp.int32),
            pltpu.VMEM((_DCH, 16), jnp.float32),
            pltpu.VMEM((_DCH, 16), jnp.float32),
            pltpu.VMEM((_DCH, 16), jnp.float32),
            pltpu.VMEM((_DCH, 16), jnp.float32),
            pltpu.SemaphoreType.DMA,
            pltpu.SemaphoreType.DMA,
        ],
    )(xyz16, src_ch, dst_ch)


def _gather_add_sc(x, table, map_ch, n_rows):
    CH = 80
    n_chunks = n_rows // CH
    kmax = -(-n_chunks // _NW)

    def body(x_hbm, tab_hbm, map_hbm, out_hbm, idx_v, x_v, g_v, sem):
        c = lax.axis_index("c")
        s = lax.axis_index("s")
        w = s * _NC + c

        def chunk(k, carry):
            cid = k * _NW + w

            @pl.when(cid < n_chunks)
            def _():
                pltpu.sync_copy(map_hbm.at[cid], idx_v)
                roff = pl.multiple_of(cid * CH, CH)
                d1 = pltpu.async_copy(tab_hbm.at[idx_v], g_v, sem)
                pltpu.sync_copy(x_hbm.at[pl.ds(roff, CH)], x_v)
                d1.wait()

                def row(r, carry2):
                    for j in range(8):
                        sl = pl.ds(j * 16, 16)
                        x_v[r, sl] = x_v[r, sl] + g_v[r, sl]
                    return carry2

                lax.fori_loop(0, CH, row, 0)
                pltpu.sync_copy(x_v, out_hbm.at[pl.ds(roff, CH)])

            return carry

        lax.fori_loop(0, kmax, chunk, 0)

    return pl.kernel(
        body,
        out_type=jax.ShapeDtypeStruct((n_rows, N_FILTERS), jnp.float32),
        mesh=_sc_mesh(),
        scratch_types=[
            pltpu.VMEM((CH,), jnp.int32),
            pltpu.VMEM((CH, N_FILTERS), jnp.float32),
            pltpu.VMEM((CH, N_FILTERS), jnp.float32),
            pltpu.SemaphoreType.DMA,
        ],
    )(x, table, map_ch)


def kernel(z, xyz, cg_xyz, mapping, nbr_list, CG_nbr_list, embed,
           W_ef1, b_ef1, W_ef2, b_ef2, W_nf, b_nf, W_u1, b_u1, W_u2, b_u2):
    z2d = z.astype(jnp.int32)[:, None]
    m2d = mapping.astype(jnp.int32)[:, None]

    src_a = nbr_list[:, 0].astype(jnp.int32).reshape(N_EDGES // _CH, _CH)
    dst_a = nbr_list[:, 1].astype(jnp.int32).reshape(N_EDGES // _CH, _CH)
    src_c = CG_nbr_list[:, 0].astype(jnp.int32).reshape(N_CG_EDGES // _CH, _CH)
    dst_c = CG_nbr_list[:, 1].astype(jnp.int32).reshape(N_CG_EDGES // _CH, _CH)
    xyz16 = jnp.zeros((N_ATOMS, 16), jnp.float32).at[:, :3].set(xyz.astype(jnp.float32))
    cg16 = jnp.zeros((N_CG, 16), jnp.float32).at[:, :3].set(cg_xyz.astype(jnp.float32))
    diff2_a = _edge_diff_sc(
        xyz16,
        src_a.reshape(N_EDGES // _DCH, _DCH),
        dst_a.reshape(N_EDGES // _DCH, _DCH))
    diff2_c = _edge_diff_sc(
        cg16,
        src_c.reshape(N_CG_EDGES // _DCH, _DCH),
        dst_c.reshape(N_CG_EDGES // _DCH, _DCH))
    map_ch = mapping.astype(jnp.int32).reshape(N_ATOMS // 80, 80)
    zeros_br = jnp.zeros((40, N_FILTERS), jnp.float32)
    zeros_a = jnp.zeros((1000, N_FILTERS), jnp.float32)
    zeros_c = jnp.zeros((200, N_FILTERS), jnp.float32)

    s_i = _embed(z2d, embed)
    counts = _counts(m2d)
    c_col = counts.T

    S_I = None
    f = _edge_filter(diff2_a, W_ef1[0], b_ef1[0][None, :], W_ef2[0], b_ef2[0][None, :])
    for i in range(N_CONV):
        j = N_CONV + i
        rn = _rn(s_i, W_nf[i], b_nf[i][None, :])
        agg2 = _msg_scatter_sc(rn, f, src_a, dst_a, zeros_a, N_ATOMS, N_EDGES // _CH)
        fc = _edge_filter(diff2_c, W_ef1[j], b_ef1[j][None, :], W_ef2[j], b_ef2[j][None, :])
        if i < N_CONV - 1:
            f = _edge_filter(diff2_a, W_ef1[i + 1], b_ef1[i + 1][None, :], W_ef2[i + 1], b_ef2[i + 1][None, :])
        s_i = _update(s_i, agg2[0], agg2[1], W_u1[i], b_u1[i][None, :], W_u2[i], b_u2[i][None, :])

        S_parts = _seg_sum_sc(s_i, map_ch, zeros_br, N_ATOMS, N_CG)
        S_input = _div(S_parts[0], S_parts[1], c_col)
        if i == 0:
            S_I = S_input

        Rn = _rn(S_input, W_nf[j], b_nf[j][None, :])
        Agg2 = _msg_scatter_sc(Rn, fc, src_c, dst_c, zeros_c, N_CG, N_CG_EDGES // _CH)
        S_I = _update(S_I, Agg2[0], Agg2[1], W_u1[j], b_u1[j][None, :], W_u2[j], b_u2[j][None, :])

        if i < N_CONV - 1:
            s_i = _gather_add_sc(s_i, S_I, map_ch, N_ATOMS)

    return S_I
